# Initial kernel scaffold; baseline (speedup 1.0000x reference)
import jax
import jax.numpy as jnp
from jax.experimental import pallas as pl


def kernel(x, edge_index, params):
    raise NotImplementedError("write your pallas kernel here")


# speedup vs baseline: 6.5031x; 6.5031x over previous
import functools

import jax
import jax.numpy as jnp
from jax import lax
from jax.experimental import pallas as pl
from jax.experimental.pallas import tpu as pltpu
from jax.experimental.pallas import tpu_sc as plsc

N = 10000
E = 160000
D_IN = 256
HID = 128
HEADS = 8
N_CLS = 40

N_PAD = 10240
E_PAD = 163840
DUMMY = N_PAD - 1
NW = 32
CHUNK = 128
W_CHUNKS = E_PAD // (NW * CHUNK)
ROWS_PER_TILE = N_PAD // 16

BM = 512
GRID_M = N_PAD // BM

f32 = jnp.float32


def _zero_acc(zrow_hbm, acc, s):
    pltpu.sync_copy(zrow_hbm, acc.at[pl.ds(s * ROWS_PER_TILE, ROWS_PER_TILE)])


@functools.cache
def _sc_kernels():
  mesh = plsc.VectorSubcoreMesh(core_axis_name="c", subcore_axis_name="s")

  @functools.partial(
      pl.kernel, mesh=mesh,
      out_type=jax.ShapeDtypeStruct((2, N_PAD, 128), f32),
      scratch_types=[
          pltpu.VMEM((W_CHUNKS, CHUNK), jnp.int32),
          pltpu.VMEM((CHUNK, 128), f32),
          pltpu.VMEM_SHARED((N_PAD, 128), f32),
      ],
  )
  def _sc_degree(cols_hbm, ones_hbm, zrow_hbm, out_hbm, colv, obuf, acc):
    c = lax.axis_index("c")
    s = lax.axis_index("s")
    w = s * 2 + c
    pltpu.sync_copy(cols_hbm.at[w], colv)
    pltpu.sync_copy(ones_hbm, obuf)
    _zero_acc(zrow_hbm, acc, s)
    plsc.subcore_barrier()

    def body(j, carry):
        pltpu.sync_copy(obuf, acc.at[colv.at[j]], add=True)
        return carry
    lax.fori_loop(0, W_CHUNKS, body, 0)
    plsc.subcore_barrier()
    pltpu.sync_copy(
        acc.at[pl.ds(s * ROWS_PER_TILE, ROWS_PER_TILE)],
        out_hbm.at[c, pl.ds(s * ROWS_PER_TILE, ROWS_PER_TILE)],
    )

  @functools.partial(
      pl.kernel, mesh=mesh,
      out_type=[
          jax.ShapeDtypeStruct((3, 2, N_PAD, 128), f32),
          jax.ShapeDtypeStruct((2, N_PAD, 128), f32),
          jax.ShapeDtypeStruct((NW * W_CHUNKS, 16, 128), f32),
      ],
      scratch_types=[
          pltpu.VMEM((W_CHUNKS, CHUNK), jnp.int32),
          pltpu.VMEM((W_CHUNKS, CHUNK), jnp.int32),
          pltpu.VMEM((CHUNK, 128), f32),
          pltpu.VMEM((CHUNK, 128), f32),
          pltpu.VMEM((16, 128), f32),
          pltpu.VMEM((1, 16), f32),
          pltpu.VMEM_SHARED((N_PAD, 128), f32),
          pltpu.SemaphoreType.DMA,
          pltpu.SemaphoreType.DMA,
      ],
  )
  def _sc_edge_pass(f0_hbm, f1_hbm, f2_hbm, asrc_hbm, adst_hbm,
                    rows_hbm, cols_hbm, cmax_hbm, zrow_hbm,
                    feat_out, den_out, alpha_out,
                    rowv, colv, gbuf, dbuf, albuf, cbuf, acc, sem_a, sem_b):
    c = lax.axis_index("c")
    s = lax.axis_index("s")
    w = s * 2 + c
    pltpu.sync_copy(rows_hbm.at[w], rowv)
    pltpu.sync_copy(cols_hbm.at[w], colv)
    pltpu.sync_copy(cmax_hbm, cbuf)

    for p, f_hbm in enumerate((f0_hbm, f1_hbm, f2_hbm)):
        _zero_acc(zrow_hbm, acc, s)
        plsc.subcore_barrier()

        def body(j, carry):
            pltpu.async_copy(f_hbm.at[rowv.at[j]], gbuf, sem_a).wait()
            pltpu.sync_copy(gbuf, acc.at[colv.at[j]], add=True)
            return carry
        lax.fori_loop(0, W_CHUNKS, body, 0)
        plsc.subcore_barrier()
        pltpu.sync_copy(
            acc.at[pl.ds(s * ROWS_PER_TILE, ROWS_PER_TILE)],
            feat_out.at[p, c, pl.ds(s * ROWS_PER_TILE, ROWS_PER_TILE)],
        )
        plsc.subcore_barrier()

    _zero_acc(zrow_hbm, acc, s)
    plsc.subcore_barrier()

    def body_a(j, carry):
        pltpu.async_copy(asrc_hbm.at[rowv.at[j]], gbuf, sem_a).wait()
        pltpu.async_copy(adst_hbm.at[colv.at[j]], dbuf, sem_b).wait()
        cv = cbuf[0]

        def edge(e, carry2):
            sv = gbuf[e, pl.ds(0, 16)] + dbuf[e, pl.ds(0, 16)]
            sv = jnp.where(sv > 0.0, sv, sv * 0.2)
            al = jnp.exp(sv - cv)
            dbuf[e, pl.ds(0, 16)] = al
            albuf[e // 8, pl.ds((e % 8) * 16, 16)] = al
            return carry2
        lax.fori_loop(0, CHUNK, edge, 0)
        pltpu.sync_copy(albuf, alpha_out.at[w * W_CHUNKS + j])
        pltpu.sync_copy(dbuf, acc.at[colv.at[j]], add=True)
        return carry
    lax.fori_loop(0, W_CHUNKS, body_a, 0)
    plsc.subcore_barrier()
    pltpu.sync_copy(
        acc.at[pl.ds(s * ROWS_PER_TILE, ROWS_PER_TILE)],
        den_out.at[c, pl.ds(s * ROWS_PER_TILE, ROWS_PER_TILE)],
    )

  @functools.partial(
      pl.kernel, mesh=mesh,
      out_type=jax.ShapeDtypeStruct((HEADS, N_PAD, HID), f32),
      scratch_types=[
          pltpu.VMEM((W_CHUNKS, CHUNK), jnp.int32),
          pltpu.VMEM((W_CHUNKS, CHUNK), jnp.int32),
          pltpu.VMEM((16, 128), f32),
          pltpu.VMEM((CHUNK, HID), f32),
          pltpu.VMEM_SHARED((N_PAD, HID), f32),
          pltpu.SemaphoreType.DMA,
      ],
  )
  def _sc_gat_msg(xsrc8_hbm, alpha_hbm, rows_hbm, cols_hbm, zrow_hbm,
                  out_hbm, rowv, colv, albuf, xbuf, acc, sem_x):
    c = lax.axis_index("c")
    s = lax.axis_index("s")

    for cval in range(2):
        @pl.when(c == cval)
        def _per_core():
            for h in range(4):
                head = cval * 4 + h
                _zero_acc(zrow_hbm, acc, s)
                plsc.subcore_barrier()
                for hh in range(2):
                    wlin = 2 * s + hh
                    pltpu.sync_copy(rows_hbm.at[wlin], rowv)
                    pltpu.sync_copy(cols_hbm.at[wlin], colv)

                    def body(j, carry):
                        pltpu.sync_copy(
                            alpha_hbm.at[wlin * W_CHUNKS + j], albuf)
                        pltpu.async_copy(
                            xsrc8_hbm.at[head].at[rowv.at[j]], xbuf, sem_x
                        ).wait()

                        def edge(e, carry2):
                            wv = albuf[e // 8, pl.ds((e % 8) * 16, 16)]
                            ws = wv[head]
                            for q in range(8):
                                xbuf[e, pl.ds(q * 16, 16)] = (
                                    ws * xbuf[e, pl.ds(q * 16, 16)])
                            return carry2
                        lax.fori_loop(0, CHUNK, edge, 0)
                        pltpu.sync_copy(xbuf, acc.at[colv.at[j]], add=True)
                        return carry
                    lax.fori_loop(0, W_CHUNKS, body, 0)
                plsc.subcore_barrier()
                pltpu.sync_copy(
                    acc.at[pl.ds(s * ROWS_PER_TILE, ROWS_PER_TILE)],
                    out_hbm.at[head, pl.ds(s * ROWS_PER_TILE, ROWS_PER_TILE)],
                )
                plsc.subcore_barrier()

  return _sc_degree, _sc_edge_pass, _sc_gat_msg


def _aug_body(x_ref, w1_ref, b1_ref, w2_ref, b2_ref, out_ref):
    xb = x_ref[...]
    t1 = jnp.tanh(jnp.dot(xb, w1_ref[...], preferred_element_type=f32)
                  + b1_ref[...])
    t2 = jax.nn.sigmoid(jnp.dot(xb, w2_ref[...], preferred_element_type=f32)
                        + b2_ref[...])
    out_ref[...] = jnp.concatenate([xb, t1, t2], axis=1)


def _tc_aug(xp, w1, b1, w2, b2):
    return pl.pallas_call(
        _aug_body,
        grid=(GRID_M,),
        in_specs=[
            pl.BlockSpec((BM, D_IN), lambda i: (i, 0)),
            pl.BlockSpec((D_IN, D_IN), lambda i: (0, 0)),
            pl.BlockSpec((1, D_IN), lambda i: (0, 0)),
            pl.BlockSpec((D_IN, D_IN), lambda i: (0, 0)),
            pl.BlockSpec((1, D_IN), lambda i: (0, 0)),
        ],
        out_specs=pl.BlockSpec((BM, 3 * D_IN), lambda i: (i, 0)),
        out_shape=jax.ShapeDtypeStruct((N_PAD, 3 * D_IN), f32),
    )(xp, w1, b1, w2, b2)


def _bigmm_body(h_ref, w_ref, gas_ref, gad_ref, dinv_ref, scale_ref, shift_ref,
                f0_ref, f1_ref, f2_ref, hr_ref, xsrc_ref, asrc_ref, adst_ref,
                *, with_bn):
    hb = h_ref[...]
    if with_bn:
        hb = jnp.maximum(hb * scale_ref[...] + shift_ref[...], 0.0)
    p = jnp.dot(hb, w_ref[...], preferred_element_type=f32)
    dinv = dinv_ref[...]
    f0_ref[...] = p[:, :HID] * dinv
    f1_ref[...] = p[:, HID:2 * HID]
    f2_ref[...] = p[:, 2 * HID:3 * HID]
    hr_ref[...] = p[:, 3 * HID:4 * HID]
    xsrc = p[:, 4 * HID:]
    xsrc_ref[...] = xsrc
    xr = xsrc.reshape(BM, HEADS, HID)
    a_s = jnp.sum(xr * gas_ref[...][None], axis=-1)
    a_d = jnp.sum(xr * gad_ref[...][None], axis=-1)
    z = jnp.zeros((BM, 120), f32)
    asrc_ref[...] = jnp.concatenate([a_s, z], axis=1)
    adst_ref[...] = jnp.concatenate([a_d, z], axis=1)


def _tc_bigmm(h, wcat, gas, gad, dinv, scale, shift, with_bn):
    ic = h.shape[1]
    wn = wcat.shape[1]
    body = functools.partial(_bigmm_body, with_bn=with_bn)
    outsp = pl.BlockSpec((BM, HID), lambda i: (i, 0))
    outsh = jax.ShapeDtypeStruct((N_PAD, HID), f32)
    return pl.pallas_call(
        body,
        grid=(GRID_M,),
        in_specs=[
            pl.BlockSpec((BM, ic), lambda i: (i, 0)),
            pl.BlockSpec((ic, wn), lambda i: (0, 0)),
            pl.BlockSpec((HEADS, HID), lambda i: (0, 0)),
            pl.BlockSpec((HEADS, HID), lambda i: (0, 0)),
            pl.BlockSpec((BM, 1), lambda i: (i, 0)),
            pl.BlockSpec((1, ic), lambda i: (0, 0)),
            pl.BlockSpec((1, ic), lambda i: (0, 0)),
        ],
        out_specs=[outsp, outsp, outsp,
                   outsp,
                   pl.BlockSpec((BM, HEADS * HID), lambda i: (i, 0)),
                   outsp, outsp],
        out_shape=[outsh, outsh, outsh,
                   outsh,
                   jax.ShapeDtypeStruct((N_PAD, HEADS * HID), f32),
                   outsh, outsh],
    )(h, wcat, gas, gad, dinv, scale, shift)


def _assemble_body(sg_ref, ss_ref, sgin_ref, m_ref, den_ref, asrc_ref,
                   adst_ref, cmax_ref, xgp_ref, xgin_ref, hr_ref,
                   xsrc_ref, dinv_ref, dci_ref, w2_ref, bias_ref,
                   hcat_ref, stats_ref):
    i = pl.program_id(0)
    bias = bias_ref[...]
    dinv = dinv_ref[...]
    gcn = dinv * (sg_ref[...] + xgp_ref[...]) + bias[0, :HID][None]
    sage = ss_ref[...] * dci_ref[...] + hr_ref[...] + bias[0, HID:2 * HID][None]
    gpre = jnp.maximum(xgin_ref[...] + sgin_ref[...]
                       + bias[0, 3 * HID:][None], 0.0)
    gin = jnp.dot(gpre, w2_ref[...], preferred_element_type=f32) \
        + bias[1, 3 * HID:][None]
    sv = asrc_ref[...] + adst_ref[...]
    sv = jnp.where(sv > 0.0, sv, sv * 0.2)
    asl = jnp.exp(sv - cmax_ref[0, 0])[:, :HEADS]
    den8 = den_ref[...][:, :HEADS] + asl
    dinv8 = 1.0 / (den8 + 1e-16)
    mr = m_ref[...].reshape(BM, HEADS, HID)
    xr = xsrc_ref[...].reshape(BM, HEADS, HID)
    gsum = jnp.sum((mr + asl[:, :, None] * xr) * dinv8[:, :, None], axis=1)
    gat = gsum * (1.0 / HEADS) + bias[0, 2 * HID:3 * HID][None]
    hcat = jnp.concatenate([gcn, sage, gat, gin], axis=1)
    hcat_ref[...] = hcat
    rowid = i * BM + lax.broadcasted_iota(jnp.int32, (BM, 1), 0)
    hm = jnp.where(rowid < N, hcat, 0.0)
    ssum = jnp.sum(hm, axis=0, keepdims=True)
    ssq = jnp.sum(hm * hm, axis=0, keepdims=True)
    blk = jnp.concatenate([ssum, ssq, jnp.zeros((6, 4 * HID), f32)], axis=0)

    @pl.when(i == 0)
    def _init():
        stats_ref[...] = blk

    @pl.when(i != 0)
    def _acc():
        stats_ref[...] = stats_ref[...] + blk


def _tc_assemble(sg, ss, sgin, m, den, asrc, adst, cmax, xgp, xgin, hr, xsrc,
                 dinv, dci, w2, bias):
    sp = pl.BlockSpec((BM, HID), lambda i: (i, 0))
    return pl.pallas_call(
        _assemble_body,
        grid=(GRID_M,),
        in_specs=[
            sp, sp, sp,
            pl.BlockSpec((BM, HEADS * HID), lambda i: (i, 0)),
            sp, sp, sp,
            pl.BlockSpec((1, 16), lambda i: (0, 0)),
            sp, sp, sp,
            pl.BlockSpec((BM, HEADS * HID), lambda i: (i, 0)),
            pl.BlockSpec((BM, 1), lambda i: (i, 0)),
            pl.BlockSpec((BM, 1), lambda i: (i, 0)),
            pl.BlockSpec((HID, HID), lambda i: (0, 0)),
            pl.BlockSpec((2, 4 * HID), lambda i: (0, 0)),
        ],
        out_specs=[
            pl.BlockSpec((BM, 4 * HID), lambda i: (i, 0)),
            pl.BlockSpec((8, 4 * HID), lambda i: (0, 0)),
        ],
        out_shape=[
            jax.ShapeDtypeStruct((N_PAD, 4 * HID), f32),
            jax.ShapeDtypeStruct((8, 4 * HID), f32),
        ],
    )(sg, ss, sgin, m, den, asrc, adst, cmax, xgp, xgin, hr, xsrc,
      dinv, dci, w2, bias)


def _final_body(h_ref, scale_ref, shift_ref, w_ref, b_ref, out_ref):
    hb = jnp.maximum(h_ref[...] * scale_ref[...] + shift_ref[...], 0.0)
    z = jnp.dot(hb, w_ref[...], preferred_element_type=f32) + b_ref[...]
    colid = lax.broadcasted_iota(jnp.int32, (BM, 128), 1)
    zm = jnp.where(colid < N_CLS, z, -jnp.inf)
    m = jnp.max(zm, axis=1, keepdims=True)
    lse = jnp.log(jnp.sum(jnp.exp(zm - m), axis=1, keepdims=True))
    out_ref[...] = (z - m - lse)[:, :N_CLS]


def _tc_final(h, scale, shift, w, b):
    return pl.pallas_call(
        _final_body,
        grid=(GRID_M,),
        in_specs=[
            pl.BlockSpec((BM, 4 * HID), lambda i: (i, 0)),
            pl.BlockSpec((1, 4 * HID), lambda i: (0, 0)),
            pl.BlockSpec((1, 4 * HID), lambda i: (0, 0)),
            pl.BlockSpec((4 * HID, 128), lambda i: (0, 0)),
            pl.BlockSpec((1, 128), lambda i: (0, 0)),
        ],
        out_specs=pl.BlockSpec((BM, N_CLS), lambda i: (i, 0)),
        out_shape=jax.ShapeDtypeStruct((N_PAD, N_CLS), f32),
    )(h, scale, shift, w, b)


def kernel(x, edge_index, params):
    p = params
    row = edge_index[0].astype(jnp.int32)
    col = edge_index[1].astype(jnp.int32)
    rowp = jnp.concatenate([row, jnp.full((E_PAD - E,), DUMMY, jnp.int32)])
    colp = jnp.concatenate([col, jnp.full((E_PAD - E,), DUMMY, jnp.int32)])
    rows32 = rowp.reshape(NW, W_CHUNKS, CHUNK)
    cols32 = colp.reshape(NW, W_CHUNKS, CHUNK)

    z128 = jnp.zeros((ROWS_PER_TILE, 128), f32)
    o128 = jnp.ones((CHUNK, 128), f32)
    sc_degree, sc_edge_pass, sc_gat_msg = _sc_kernels()

    degp = sc_degree(cols32, o128, z128)
    deg = degp[0, :, 0] + degp[1, :, 0]
    dinv = (deg + 1.0) ** -0.5
    dci = 1.0 / jnp.maximum(deg, 1.0)
    dinv2 = dinv[:, None]
    dci2 = dci[:, None]

    xp = jnp.pad(x, ((0, N_PAD - N), (0, 0)))
    h = _tc_aug(xp, p['aug_W1'], p['aug_b1'][None], p['aug_W2'],
                p['aug_b2'][None])

    scale = jnp.ones((1, 3 * D_IN), f32)
    shift = jnp.zeros((1, 3 * D_IN), f32)
    for i in range(2):
        wcat = jnp.concatenate(
            [p[f'gcn_W{i}'], p[f'sage_Wl{i}'], p[f'gin_W1{i}'],
             p[f'sage_Wr{i}'], p[f'gat_W{i}']], axis=1)
        f0, f1, f2, hr, xsrc, asrc, adst = _tc_bigmm(
            h, wcat, p[f'gat_as{i}'], p[f'gat_ad{i}'], dinv2, scale, shift,
            with_bn=(i > 0))
        cmax = jnp.max(asrc[:N, :8]) + jnp.max(adst[:N, :8])
        cmax = jnp.maximum(cmax, 0.0)
        cmax16 = jnp.full((1, 16), cmax, f32)

        s, den, alpha = sc_edge_pass(f0, f1, f2, asrc, adst,
                                      rows32, cols32, cmax16, z128)
        xsrc8 = xsrc.reshape(N_PAD, HEADS, HID).transpose(1, 0, 2)
        m8 = sc_gat_msg(xsrc8, alpha, rows32, cols32, z128)
        m = m8.transpose(1, 0, 2).reshape(N_PAD, HEADS * HID)

        bias = jnp.stack([
            jnp.concatenate([p[f'gcn_b{i}'], p[f'sage_b{i}'], p[f'gat_b{i}'],
                             p[f'gin_b1{i}']]),
            jnp.concatenate([jnp.zeros((3 * HID,), f32), p[f'gin_b2{i}']]),
        ])
        hcat, stats = _tc_assemble(
            s[0, 0] + s[0, 1], s[1, 0] + s[1, 1], s[2, 0] + s[2, 1],
            m, den[0] + den[1], asrc, adst, cmax16, f0, f2, hr, xsrc,
            dinv2, dci2, p[f'gin_W2{i}'], bias)
        mu = stats[0] / N
        var = stats[1] / N - mu * mu
        scale = (p[f'bn_g{i}'] / jnp.sqrt(var + 1e-5))[None]
        shift = (p[f'bn_b{i}'] - mu * scale[0])[None]
        h = hcat

    wout = jnp.pad(p['out_W'], ((0, 0), (0, 128 - N_CLS)))
    bout = jnp.pad(p['out_b'], ((0, 128 - N_CLS)))[None]
    out = _tc_final(h, scale, shift, wout, bout)
    return out[:N]


# speedup vs baseline: 8.5714x; 1.3181x over previous
import functools

import jax
import jax.numpy as jnp
from jax import lax
from jax.experimental import pallas as pl
from jax.experimental.pallas import tpu as pltpu
from jax.experimental.pallas import tpu_sc as plsc

N = 10000
E = 160000
D_IN = 256
HID = 128
HEADS = 8
N_CLS = 40

N_PAD = 10240
E_PAD = 163840
DUMMY = N_PAD - 1
NW = 32
CHUNK = 128
W_CHUNKS = E_PAD // (NW * CHUNK)
ROWS_PER_TILE = N_PAD // 16

BM = 512
GRID_M = N_PAD // BM

f32 = jnp.float32


def _zero_acc(zrow_hbm, acc, s):
    pltpu.sync_copy(zrow_hbm, acc.at[pl.ds(s * ROWS_PER_TILE, ROWS_PER_TILE)])


@functools.cache
def _sc_kernels():
  mesh = plsc.VectorSubcoreMesh(core_axis_name="c", subcore_axis_name="s")

  @functools.partial(
      pl.kernel, mesh=mesh,
      out_type=jax.ShapeDtypeStruct((2, N_PAD, 128), f32),
      scratch_types=[
          pltpu.VMEM((W_CHUNKS, CHUNK), jnp.int32),
          pltpu.VMEM((CHUNK, 128), f32),
          pltpu.VMEM_SHARED((N_PAD, 128), f32),
      ],
  )
  def _sc_degree(cols_hbm, ones_hbm, zrow_hbm, out_hbm, colv, obuf, acc):
    c = lax.axis_index("c")
    s = lax.axis_index("s")
    w = s * 2 + c
    pltpu.sync_copy(cols_hbm.at[w], colv)
    pltpu.sync_copy(ones_hbm, obuf)
    _zero_acc(zrow_hbm, acc, s)
    plsc.subcore_barrier()

    def body(j, carry):
        pltpu.sync_copy(obuf, acc.at[colv.at[j]], add=True)
        return carry
    lax.fori_loop(0, W_CHUNKS, body, 0)
    plsc.subcore_barrier()
    pltpu.sync_copy(
        acc.at[pl.ds(s * ROWS_PER_TILE, ROWS_PER_TILE)],
        out_hbm.at[c, pl.ds(s * ROWS_PER_TILE, ROWS_PER_TILE)],
    )

  @functools.partial(
      pl.kernel, mesh=mesh,
      out_type=[
          jax.ShapeDtypeStruct((3, 2, N_PAD, 128), f32),
          jax.ShapeDtypeStruct((2, N_PAD, 128), f32),
          jax.ShapeDtypeStruct((NW * W_CHUNKS, 16, 128), f32),
      ],
      scratch_types=[
          pltpu.VMEM((W_CHUNKS, CHUNK), jnp.int32),
          pltpu.VMEM((W_CHUNKS, CHUNK), jnp.int32),
          pltpu.VMEM((CHUNK, 128), f32),
          pltpu.VMEM((CHUNK, 128), f32),
          pltpu.VMEM((16, 128), f32),
          pltpu.VMEM((1, 16), f32),
          pltpu.VMEM_SHARED((N_PAD, 128), f32),
          pltpu.SemaphoreType.DMA,
          pltpu.SemaphoreType.DMA,
          pltpu.SemaphoreType.DMA,
          pltpu.SemaphoreType.DMA,
      ],
  )
  def _sc_edge_pass(f0_hbm, f1_hbm, f2_hbm, asrc_hbm, adst_hbm,
                    rows_hbm, cols_hbm, cmax_hbm, zrow_hbm,
                    feat_out, den_out, alpha_out,
                    rowv, colv, bufa, bufb, albuf, cbuf, acc,
                    sga, sgb, ssa, ssb):
    c = lax.axis_index("c")
    s = lax.axis_index("s")
    w = s * 2 + c
    pltpu.sync_copy(rows_hbm.at[w], rowv)
    pltpu.sync_copy(cols_hbm.at[w], colv)
    pltpu.sync_copy(cmax_hbm, cbuf)

    def s_start(j, buf, sem):
        pltpu.async_copy(buf, acc.at[colv.at[j]], sem, add=True)

    def s_wait(buf, sem):
        pltpu.make_async_copy(buf, acc.at[colv.at[0]], sem).wait()

    for p, f_hbm in enumerate((f0_hbm, f1_hbm, f2_hbm)):
        _zero_acc(zrow_hbm, acc, s)
        plsc.subcore_barrier()
        pltpu.async_copy(f_hbm.at[rowv.at[0]], bufa, sga)

        def body(j2, carry):
            for b in range(2):
                j = 2 * j2 + b
                buf, sg, ss = (bufa, sga, ssa) if b == 0 else (bufb, sgb, ssb)
                obuf, osg, oss = (bufb, sgb, ssb) if b == 0 else (bufa, sga, ssa)
                pltpu.make_async_copy(f_hbm.at[rowv.at[0]], buf, sg).wait()

                @pl.when(j + 1 < W_CHUNKS)
                def _prefetch():
                    @pl.when(j >= 1)
                    def _ws():
                        s_wait(obuf, oss)
                    pltpu.async_copy(f_hbm.at[rowv.at[j + 1]], obuf, osg)

                s_start(j, buf, ss)
            return carry
        lax.fori_loop(0, W_CHUNKS // 2, body, 0)
        s_wait(bufa, ssa)
        s_wait(bufb, ssb)
        plsc.subcore_barrier()
        pltpu.sync_copy(
            acc.at[pl.ds(s * ROWS_PER_TILE, ROWS_PER_TILE)],
            feat_out.at[p, c, pl.ds(s * ROWS_PER_TILE, ROWS_PER_TILE)],
        )
        plsc.subcore_barrier()

    _zero_acc(zrow_hbm, acc, s)
    plsc.subcore_barrier()

    def body_a(j, carry):
        @pl.when(j >= 1)
        def _ws():
            s_wait(bufb, ssb)
        pltpu.async_copy(asrc_hbm.at[rowv.at[j]], bufa, sga)
        pltpu.async_copy(adst_hbm.at[colv.at[j]], bufb, sgb)
        pltpu.make_async_copy(asrc_hbm.at[rowv.at[0]], bufa, sga).wait()
        pltpu.make_async_copy(adst_hbm.at[colv.at[0]], bufb, sgb).wait()
        cv = cbuf[0]

        def edge(e, carry2):
            sv = bufa[e, pl.ds(0, 16)] + bufb[e, pl.ds(0, 16)]
            sv = jnp.where(sv > 0.0, sv, sv * 0.2)
            al = jnp.exp(sv - cv)
            bufb[e, pl.ds(0, 16)] = al
            albuf[e // 8, pl.ds((e % 8) * 16, 16)] = al
            return carry2
        lax.fori_loop(0, CHUNK, edge, 0)
        pltpu.sync_copy(albuf, alpha_out.at[w * W_CHUNKS + j])
        s_start(j, bufb, ssb)
        return carry
    lax.fori_loop(0, W_CHUNKS, body_a, 0)
    s_wait(bufb, ssb)
    plsc.subcore_barrier()
    pltpu.sync_copy(
        acc.at[pl.ds(s * ROWS_PER_TILE, ROWS_PER_TILE)],
        den_out.at[c, pl.ds(s * ROWS_PER_TILE, ROWS_PER_TILE)],
    )

  @functools.partial(
      pl.kernel, mesh=mesh,
      out_type=jax.ShapeDtypeStruct((HEADS, N_PAD, HID), f32),
      scratch_types=[
          pltpu.VMEM((W_CHUNKS, CHUNK), jnp.int32),
          pltpu.VMEM((W_CHUNKS, CHUNK), jnp.int32),
          pltpu.VMEM((2, 16, 128), f32),
          pltpu.VMEM((CHUNK, HID), f32),
          pltpu.VMEM((CHUNK, HID), f32),
          pltpu.VMEM_SHARED((N_PAD, HID), f32),
          pltpu.SemaphoreType.DMA,
          pltpu.SemaphoreType.DMA,
          pltpu.SemaphoreType.DMA,
          pltpu.SemaphoreType.DMA,
          pltpu.SemaphoreType.DMA,
          pltpu.SemaphoreType.DMA,
      ],
  )
  def _sc_gat_msg(xsrc8_hbm, alpha_hbm, rows_hbm, cols_hbm, zrow_hbm,
                  out_hbm, rowv, colv, albuf, xbufa, xbufb, acc,
                  sala, salb, sxa, sxb, ssa, ssb):
    c = lax.axis_index("c")
    s = lax.axis_index("s")

    def s_wait(buf, sem):
        pltpu.make_async_copy(buf, acc.at[colv.at[0]], sem).wait()

    for cval in range(2):
        @pl.when(c == cval)
        def _per_core():
            for h in range(4):
                head = cval * 4 + h
                _zero_acc(zrow_hbm, acc, s)
                plsc.subcore_barrier()
                for hh in range(2):
                    wlin = 2 * s + hh
                    pltpu.sync_copy(rows_hbm.at[wlin], rowv)
                    pltpu.sync_copy(cols_hbm.at[wlin], colv)
                    pltpu.async_copy(
                        alpha_hbm.at[wlin * W_CHUNKS], albuf.at[0], sala)
                    pltpu.async_copy(
                        xsrc8_hbm.at[head].at[rowv.at[0]], xbufa, sxa)

                    def body(j2, carry):
                        for b in range(2):
                            j = 2 * j2 + b
                            xb, sal, sx, ss = (
                                (xbufa, sala, sxa, ssa) if b == 0
                                else (xbufb, salb, sxb, ssb))
                            oxb, osal, osx, oss = (
                                (xbufb, salb, sxb, ssb) if b == 0
                                else (xbufa, sala, sxa, ssa))
                            ob = 1 - b
                            pltpu.make_async_copy(
                                alpha_hbm.at[0], albuf.at[b], sal).wait()
                            pltpu.make_async_copy(
                                xsrc8_hbm.at[0].at[rowv.at[0]], xb, sx).wait()

                            @pl.when(j + 1 < W_CHUNKS)
                            def _prefetch():
                                @pl.when(j >= 1)
                                def _ws():
                                    s_wait(oxb, oss)
                                pltpu.async_copy(
                                    alpha_hbm.at[wlin * W_CHUNKS + j + 1],
                                    albuf.at[ob], osal)
                                pltpu.async_copy(
                                    xsrc8_hbm.at[head].at[rowv.at[j + 1]],
                                    oxb, osx)

                            def edge(e, carry2):
                                wv = albuf[b, e // 8, pl.ds((e % 8) * 16, 16)]
                                ws = wv[head]
                                for q in range(8):
                                    xb[e, pl.ds(q * 16, 16)] = (
                                        ws * xb[e, pl.ds(q * 16, 16)])
                                return carry2
                            lax.fori_loop(0, CHUNK, edge, 0)
                            pltpu.async_copy(
                                xb, acc.at[colv.at[j]], ss, add=True)
                        return carry
                    lax.fori_loop(0, W_CHUNKS // 2, body, 0)
                    s_wait(xbufa, ssa)
                    s_wait(xbufb, ssb)
                plsc.subcore_barrier()
                pltpu.sync_copy(
                    acc.at[pl.ds(s * ROWS_PER_TILE, ROWS_PER_TILE)],
                    out_hbm.at[head, pl.ds(s * ROWS_PER_TILE, ROWS_PER_TILE)],
                )
                plsc.subcore_barrier()

  return _sc_degree, _sc_edge_pass, _sc_gat_msg


def _aug_body(x_ref, w1_ref, b1_ref, w2_ref, b2_ref, out_ref):
    xb = x_ref[...]
    t1 = jnp.tanh(jnp.dot(xb, w1_ref[...], preferred_element_type=f32)
                  + b1_ref[...])
    t2 = jax.nn.sigmoid(jnp.dot(xb, w2_ref[...], preferred_element_type=f32)
                        + b2_ref[...])
    out_ref[...] = jnp.concatenate([xb, t1, t2], axis=1)


def _tc_aug(xp, w1, b1, w2, b2):
    return pl.pallas_call(
        _aug_body,
        grid=(GRID_M,),
        in_specs=[
            pl.BlockSpec((BM, D_IN), lambda i: (i, 0)),
            pl.BlockSpec((D_IN, D_IN), lambda i: (0, 0)),
            pl.BlockSpec((1, D_IN), lambda i: (0, 0)),
            pl.BlockSpec((D_IN, D_IN), lambda i: (0, 0)),
            pl.BlockSpec((1, D_IN), lambda i: (0, 0)),
        ],
        out_specs=pl.BlockSpec((BM, 3 * D_IN), lambda i: (i, 0)),
        out_shape=jax.ShapeDtypeStruct((N_PAD, 3 * D_IN), f32),
    )(xp, w1, b1, w2, b2)


def _bigmm_body(h_ref, w_ref, gas_ref, gad_ref, dinv_ref, scale_ref, shift_ref,
                f0_ref, f1_ref, f2_ref, hr_ref, xsrc_ref, asrc_ref, adst_ref,
                *, with_bn):
    hb = h_ref[...]
    if with_bn:
        hb = jnp.maximum(hb * scale_ref[...] + shift_ref[...], 0.0)
    p = jnp.dot(hb, w_ref[...], preferred_element_type=f32)
    dinv = dinv_ref[...]
    f0_ref[...] = p[:, :HID] * dinv
    f1_ref[...] = p[:, HID:2 * HID]
    f2_ref[...] = p[:, 2 * HID:3 * HID]
    hr_ref[...] = p[:, 3 * HID:4 * HID]
    xsrc = p[:, 4 * HID:]
    xsrc_ref[...] = xsrc
    xr = xsrc.reshape(BM, HEADS, HID)
    a_s = jnp.sum(xr * gas_ref[...][None], axis=-1)
    a_d = jnp.sum(xr * gad_ref[...][None], axis=-1)
    z = jnp.zeros((BM, 120), f32)
    asrc_ref[...] = jnp.concatenate([a_s, z], axis=1)
    adst_ref[...] = jnp.concatenate([a_d, z], axis=1)


def _tc_bigmm(h, wcat, gas, gad, dinv, scale, shift, with_bn):
    ic = h.shape[1]
    wn = wcat.shape[1]
    body = functools.partial(_bigmm_body, with_bn=with_bn)
    outsp = pl.BlockSpec((BM, HID), lambda i: (i, 0))
    outsh = jax.ShapeDtypeStruct((N_PAD, HID), f32)
    return pl.pallas_call(
        body,
        grid=(GRID_M,),
        in_specs=[
            pl.BlockSpec((BM, ic), lambda i: (i, 0)),
            pl.BlockSpec((ic, wn), lambda i: (0, 0)),
            pl.BlockSpec((HEADS, HID), lambda i: (0, 0)),
            pl.BlockSpec((HEADS, HID), lambda i: (0, 0)),
            pl.BlockSpec((BM, 1), lambda i: (i, 0)),
            pl.BlockSpec((1, ic), lambda i: (0, 0)),
            pl.BlockSpec((1, ic), lambda i: (0, 0)),
        ],
        out_specs=[outsp, outsp, outsp,
                   outsp,
                   pl.BlockSpec((BM, HEADS * HID), lambda i: (i, 0)),
                   outsp, outsp],
        out_shape=[outsh, outsh, outsh,
                   outsh,
                   jax.ShapeDtypeStruct((N_PAD, HEADS * HID), f32),
                   outsh, outsh],
    )(h, wcat, gas, gad, dinv, scale, shift)


def _assemble_body(sg_ref, ss_ref, sgin_ref, m_ref, den_ref, asrc_ref,
                   adst_ref, cmax_ref, xgp_ref, xgin_ref, hr_ref,
                   xsrc_ref, dinv_ref, dci_ref, w2_ref, bias_ref,
                   hcat_ref, stats_ref):
    i = pl.program_id(0)
    bias = bias_ref[...]
    dinv = dinv_ref[...]
    gcn = dinv * (sg_ref[...] + xgp_ref[...]) + bias[0, :HID][None]
    sage = ss_ref[...] * dci_ref[...] + hr_ref[...] + bias[0, HID:2 * HID][None]
    gpre = jnp.maximum(xgin_ref[...] + sgin_ref[...]
                       + bias[0, 3 * HID:][None], 0.0)
    gin = jnp.dot(gpre, w2_ref[...], preferred_element_type=f32) \
        + bias[1, 3 * HID:][None]
    sv = asrc_ref[...] + adst_ref[...]
    sv = jnp.where(sv > 0.0, sv, sv * 0.2)
    asl = jnp.exp(sv - cmax_ref[0, 0])[:, :HEADS]
    den8 = den_ref[...][:, :HEADS] + asl
    dinv8 = 1.0 / (den8 + 1e-16)
    mr = m_ref[...].reshape(BM, HEADS, HID)
    xr = xsrc_ref[...].reshape(BM, HEADS, HID)
    gsum = jnp.sum((mr + asl[:, :, None] * xr) * dinv8[:, :, None], axis=1)
    gat = gsum * (1.0 / HEADS) + bias[0, 2 * HID:3 * HID][None]
    hcat = jnp.concatenate([gcn, sage, gat, gin], axis=1)
    hcat_ref[...] = hcat
    rowid = i * BM + lax.broadcasted_iota(jnp.int32, (BM, 1), 0)
    hm = jnp.where(rowid < N, hcat, 0.0)
    ssum = jnp.sum(hm, axis=0, keepdims=True)
    ssq = jnp.sum(hm * hm, axis=0, keepdims=True)
    blk = jnp.concatenate([ssum, ssq, jnp.zeros((6, 4 * HID), f32)], axis=0)

    @pl.when(i == 0)
    def _init():
        stats_ref[...] = blk

    @pl.when(i != 0)
    def _acc():
        stats_ref[...] = stats_ref[...] + blk


def _tc_assemble(sg, ss, sgin, m, den, asrc, adst, cmax, xgp, xgin, hr, xsrc,
                 dinv, dci, w2, bias):
    sp = pl.BlockSpec((BM, HID), lambda i: (i, 0))
    return pl.pallas_call(
        _assemble_body,
        grid=(GRID_M,),
        in_specs=[
            sp, sp, sp,
            pl.BlockSpec((BM, HEADS * HID), lambda i: (i, 0)),
            sp, sp, sp,
            pl.BlockSpec((1, 16), lambda i: (0, 0)),
            sp, sp, sp,
            pl.BlockSpec((BM, HEADS * HID), lambda i: (i, 0)),
            pl.BlockSpec((BM, 1), lambda i: (i, 0)),
            pl.BlockSpec((BM, 1), lambda i: (i, 0)),
            pl.BlockSpec((HID, HID), lambda i: (0, 0)),
            pl.BlockSpec((2, 4 * HID), lambda i: (0, 0)),
        ],
        out_specs=[
            pl.BlockSpec((BM, 4 * HID), lambda i: (i, 0)),
            pl.BlockSpec((8, 4 * HID), lambda i: (0, 0)),
        ],
        out_shape=[
            jax.ShapeDtypeStruct((N_PAD, 4 * HID), f32),
            jax.ShapeDtypeStruct((8, 4 * HID), f32),
        ],
    )(sg, ss, sgin, m, den, asrc, adst, cmax, xgp, xgin, hr, xsrc,
      dinv, dci, w2, bias)


def _final_body(h_ref, scale_ref, shift_ref, w_ref, b_ref, out_ref):
    hb = jnp.maximum(h_ref[...] * scale_ref[...] + shift_ref[...], 0.0)
    z = jnp.dot(hb, w_ref[...], preferred_element_type=f32) + b_ref[...]
    colid = lax.broadcasted_iota(jnp.int32, (BM, 128), 1)
    zm = jnp.where(colid < N_CLS, z, -jnp.inf)
    m = jnp.max(zm, axis=1, keepdims=True)
    lse = jnp.log(jnp.sum(jnp.exp(zm - m), axis=1, keepdims=True))
    out_ref[...] = (z - m - lse)[:, :N_CLS]


def _tc_final(h, scale, shift, w, b):
    return pl.pallas_call(
        _final_body,
        grid=(GRID_M,),
        in_specs=[
            pl.BlockSpec((BM, 4 * HID), lambda i: (i, 0)),
            pl.BlockSpec((1, 4 * HID), lambda i: (0, 0)),
            pl.BlockSpec((1, 4 * HID), lambda i: (0, 0)),
            pl.BlockSpec((4 * HID, 128), lambda i: (0, 0)),
            pl.BlockSpec((1, 128), lambda i: (0, 0)),
        ],
        out_specs=pl.BlockSpec((BM, N_CLS), lambda i: (i, 0)),
        out_shape=jax.ShapeDtypeStruct((N_PAD, N_CLS), f32),
    )(h, scale, shift, w, b)


def kernel(x, edge_index, params):
    p = params
    row = edge_index[0].astype(jnp.int32)
    col = edge_index[1].astype(jnp.int32)
    rowp = jnp.concatenate([row, jnp.full((E_PAD - E,), DUMMY, jnp.int32)])
    colp = jnp.concatenate([col, jnp.full((E_PAD - E,), DUMMY, jnp.int32)])
    rows32 = rowp.reshape(NW, W_CHUNKS, CHUNK)
    cols32 = colp.reshape(NW, W_CHUNKS, CHUNK)

    z128 = jnp.zeros((ROWS_PER_TILE, 128), f32)
    o128 = jnp.ones((CHUNK, 128), f32)
    sc_degree, sc_edge_pass, sc_gat_msg = _sc_kernels()

    degp = sc_degree(cols32, o128, z128)
    deg = degp[0, :, 0] + degp[1, :, 0]
    dinv = (deg + 1.0) ** -0.5
    dci = 1.0 / jnp.maximum(deg, 1.0)
    dinv2 = dinv[:, None]
    dci2 = dci[:, None]

    xp = jnp.pad(x, ((0, N_PAD - N), (0, 0)))
    h = _tc_aug(xp, p['aug_W1'], p['aug_b1'][None], p['aug_W2'],
                p['aug_b2'][None])

    scale = jnp.ones((1, 3 * D_IN), f32)
    shift = jnp.zeros((1, 3 * D_IN), f32)
    for i in range(2):
        wcat = jnp.concatenate(
            [p[f'gcn_W{i}'], p[f'sage_Wl{i}'], p[f'gin_W1{i}'],
             p[f'sage_Wr{i}'], p[f'gat_W{i}']], axis=1)
        f0, f1, f2, hr, xsrc, asrc, adst = _tc_bigmm(
            h, wcat, p[f'gat_as{i}'], p[f'gat_ad{i}'], dinv2, scale, shift,
            with_bn=(i > 0))
        cmax = jnp.max(asrc[:N, :8]) + jnp.max(adst[:N, :8])
        cmax = jnp.maximum(cmax, 0.0)
        cmax16 = jnp.full((1, 16), cmax, f32)

        s, den, alpha = sc_edge_pass(f0, f1, f2, asrc, adst,
                                      rows32, cols32, cmax16, z128)
        xsrc8 = xsrc.reshape(N_PAD, HEADS, HID).transpose(1, 0, 2)
        m8 = sc_gat_msg(xsrc8, alpha, rows32, cols32, z128)
        m = m8.transpose(1, 0, 2).reshape(N_PAD, HEADS * HID)

        bias = jnp.stack([
            jnp.concatenate([p[f'gcn_b{i}'], p[f'sage_b{i}'], p[f'gat_b{i}'],
                             p[f'gin_b1{i}']]),
            jnp.concatenate([jnp.zeros((3 * HID,), f32), p[f'gin_b2{i}']]),
        ])
        hcat, stats = _tc_assemble(
            s[0, 0] + s[0, 1], s[1, 0] + s[1, 1], s[2, 0] + s[2, 1],
            m, den[0] + den[1], asrc, adst, cmax16, f0, f2, hr, xsrc,
            dinv2, dci2, p[f'gin_W2{i}'], bias)
        mu = stats[0] / N
        var = stats[1] / N - mu * mu
        scale = (p[f'bn_g{i}'] / jnp.sqrt(var + 1e-5))[None]
        shift = (p[f'bn_b{i}'] - mu * scale[0])[None]
        h = hcat

    wout = jnp.pad(p['out_W'], ((0, 0), (0, 128 - N_CLS)))
    bout = jnp.pad(p['out_b'], ((0, 128 - N_CLS)))[None]
    out = _tc_final(h, scale, shift, wout, bout)
    return out[:N]


# speedup vs baseline: 17.2107x; 2.0079x over previous
import functools

import jax
import jax.numpy as jnp
from jax import lax
from jax.experimental import pallas as pl
from jax.experimental.pallas import tpu as pltpu
from jax.experimental.pallas import tpu_sc as plsc

N = 10000
E = 160000
D_IN = 256
HID = 128
HEADS = 8
N_CLS = 40

N_PAD = 10240
E_PAD = 163840
DUMMY = N_PAD - 1
NW = 32
CHUNK = 128
W_CHUNKS = E_PAD // (NW * CHUNK)
ROWS_PER_TILE = N_PAD // 16

BM = 512
GRID_M = N_PAD // BM

f32 = jnp.float32


def _zero_acc(zrow_hbm, acc, s):
    pltpu.sync_copy(zrow_hbm, acc.at[pl.ds(s * ROWS_PER_TILE, ROWS_PER_TILE)])


@functools.cache
def _sc_kernels():
  mesh = plsc.VectorSubcoreMesh(core_axis_name="c", subcore_axis_name="s")

  @functools.partial(
      pl.kernel, mesh=mesh,
      out_type=jax.ShapeDtypeStruct((2, N_PAD, 128), f32),
      scratch_types=[
          pltpu.VMEM((W_CHUNKS, CHUNK), jnp.int32),
          pltpu.VMEM((CHUNK, 128), f32),
          pltpu.VMEM_SHARED((N_PAD, 128), f32),
      ],
  )
  def _sc_degree(cols_hbm, ones_hbm, zrow_hbm, out_hbm, colv, obuf, acc):
    c = lax.axis_index("c")
    s = lax.axis_index("s")
    w = s * 2 + c
    pltpu.sync_copy(cols_hbm.at[w], colv)
    pltpu.sync_copy(ones_hbm, obuf)
    _zero_acc(zrow_hbm, acc, s)
    plsc.subcore_barrier()

    def body(j, carry):
        pltpu.sync_copy(obuf, acc.at[colv.at[j]], add=True)
        return carry
    lax.fori_loop(0, W_CHUNKS, body, 0)
    plsc.subcore_barrier()
    pltpu.sync_copy(
        acc.at[pl.ds(s * ROWS_PER_TILE, ROWS_PER_TILE)],
        out_hbm.at[c, pl.ds(s * ROWS_PER_TILE, ROWS_PER_TILE)],
    )

  @functools.partial(
      pl.kernel, mesh=mesh,
      out_type=[
          jax.ShapeDtypeStruct((3, 2, N_PAD, 128), f32),
          jax.ShapeDtypeStruct((2, N_PAD, 128), f32),
          jax.ShapeDtypeStruct((NW * W_CHUNKS, 16, 128), f32),
      ],
      scratch_types=[
          pltpu.VMEM((W_CHUNKS, CHUNK), jnp.int32),
          pltpu.VMEM((W_CHUNKS, CHUNK), jnp.int32),
          pltpu.VMEM((CHUNK, 128), f32),
          pltpu.VMEM((CHUNK, 128), f32),
          pltpu.VMEM((16, 128), f32),
          pltpu.VMEM((1, 16), f32),
          pltpu.VMEM_SHARED((N_PAD, 128), f32),
          pltpu.SemaphoreType.DMA,
          pltpu.SemaphoreType.DMA,
          pltpu.SemaphoreType.DMA,
          pltpu.SemaphoreType.DMA,
      ],
  )
  def _sc_edge_pass(f0_hbm, f1_hbm, f2_hbm, asrc_hbm, adst_hbm,
                    rows_hbm, cols_hbm, cmax_hbm, zrow_hbm,
                    feat_out, den_out, alpha_out,
                    rowv, colv, bufa, bufb, albuf, cbuf, acc,
                    sga, sgb, ssa, ssb):
    c = lax.axis_index("c")
    s = lax.axis_index("s")
    w = s * 2 + c
    pltpu.sync_copy(rows_hbm.at[w], rowv)
    pltpu.sync_copy(cols_hbm.at[w], colv)
    pltpu.sync_copy(cmax_hbm, cbuf)

    def s_start(j, buf, sem):
        pltpu.async_copy(buf, acc.at[colv.at[j]], sem, add=True)

    def s_wait(buf, sem):
        pltpu.make_async_copy(buf, acc.at[colv.at[0]], sem).wait()

    for p, f_hbm in enumerate((f0_hbm, f1_hbm, f2_hbm)):
        _zero_acc(zrow_hbm, acc, s)
        plsc.subcore_barrier()
        pltpu.async_copy(f_hbm.at[rowv.at[0]], bufa, sga)

        def body(j2, carry):
            for b in range(2):
                j = 2 * j2 + b
                buf, sg, ss = (bufa, sga, ssa) if b == 0 else (bufb, sgb, ssb)
                obuf, osg, oss = (bufb, sgb, ssb) if b == 0 else (bufa, sga, ssa)
                pltpu.make_async_copy(f_hbm.at[rowv.at[0]], buf, sg).wait()

                @pl.when(j + 1 < W_CHUNKS)
                def _prefetch():
                    @pl.when(j >= 1)
                    def _ws():
                        s_wait(obuf, oss)
                    pltpu.async_copy(f_hbm.at[rowv.at[j + 1]], obuf, osg)

                s_start(j, buf, ss)
            return carry
        lax.fori_loop(0, W_CHUNKS // 2, body, 0)
        s_wait(bufa, ssa)
        s_wait(bufb, ssb)
        plsc.subcore_barrier()
        pltpu.sync_copy(
            acc.at[pl.ds(s * ROWS_PER_TILE, ROWS_PER_TILE)],
            feat_out.at[p, c, pl.ds(s * ROWS_PER_TILE, ROWS_PER_TILE)],
        )
        plsc.subcore_barrier()

    _zero_acc(zrow_hbm, acc, s)
    plsc.subcore_barrier()

    def body_a(j, carry):
        @pl.when(j >= 1)
        def _ws():
            s_wait(bufb, ssb)
        pltpu.async_copy(asrc_hbm.at[rowv.at[j]], bufa, sga)
        pltpu.async_copy(adst_hbm.at[colv.at[j]], bufb, sgb)
        pltpu.make_async_copy(asrc_hbm.at[rowv.at[0]], bufa, sga).wait()
        pltpu.make_async_copy(adst_hbm.at[colv.at[0]], bufb, sgb).wait()
        cv = cbuf[0]

        def edge(e, carry2):
            sv = bufa[e, pl.ds(0, 16)] + bufb[e, pl.ds(0, 16)]
            sv = jnp.where(sv > 0.0, sv, sv * 0.2)
            al = jnp.exp(sv - cv)
            bufb[e, pl.ds(0, 16)] = al
            albuf[e // 8, pl.ds((e % 8) * 16, 16)] = al
            return carry2
        lax.fori_loop(0, CHUNK, edge, 0)
        pltpu.sync_copy(albuf, alpha_out.at[w * W_CHUNKS + j])
        s_start(j, bufb, ssb)
        return carry
    lax.fori_loop(0, W_CHUNKS, body_a, 0)
    s_wait(bufb, ssb)
    plsc.subcore_barrier()
    pltpu.sync_copy(
        acc.at[pl.ds(s * ROWS_PER_TILE, ROWS_PER_TILE)],
        den_out.at[c, pl.ds(s * ROWS_PER_TILE, ROWS_PER_TILE)],
    )

  @functools.partial(
      pl.kernel, mesh=mesh,
      out_type=jax.ShapeDtypeStruct((HEADS, N_PAD, HID), f32),
      scratch_types=[
          pltpu.VMEM((W_CHUNKS, CHUNK), jnp.int32),
          pltpu.VMEM((W_CHUNKS, CHUNK), jnp.int32),
          pltpu.VMEM((2, 16, 128), f32),
          pltpu.VMEM((CHUNK, HID), f32),
          pltpu.VMEM((CHUNK, HID), f32),
          pltpu.VMEM_SHARED((N_PAD, HID), f32),
          pltpu.SemaphoreType.DMA,
          pltpu.SemaphoreType.DMA,
          pltpu.SemaphoreType.DMA,
          pltpu.SemaphoreType.DMA,
          pltpu.SemaphoreType.DMA,
          pltpu.SemaphoreType.DMA,
      ],
  )
  def _sc_gat_msg(xsrc8_hbm, alpha_hbm, rows_hbm, cols_hbm, zrow_hbm,
                  out_hbm, rowv, colv, albuf, xbufa, xbufb, acc,
                  sala, salb, sxa, sxb, ssa, ssb):
    c = lax.axis_index("c")
    s = lax.axis_index("s")

    def s_wait(buf, sem):
        pltpu.make_async_copy(buf, acc.at[colv.at[0]], sem).wait()

    for cval in range(2):
        @pl.when(c == cval)
        def _per_core():
            for h in range(4):
                head = cval * 4 + h
                _zero_acc(zrow_hbm, acc, s)
                plsc.subcore_barrier()
                for hh in range(2):
                    wlin = 2 * s + hh
                    pltpu.sync_copy(rows_hbm.at[wlin], rowv)
                    pltpu.sync_copy(cols_hbm.at[wlin], colv)
                    pltpu.async_copy(
                        alpha_hbm.at[wlin * W_CHUNKS], albuf.at[0], sala)
                    pltpu.async_copy(
                        xsrc8_hbm.at[head].at[rowv.at[0]], xbufa, sxa)

                    def body(j2, carry):
                        for b in range(2):
                            j = 2 * j2 + b
                            xb, sal, sx, ss = (
                                (xbufa, sala, sxa, ssa) if b == 0
                                else (xbufb, salb, sxb, ssb))
                            oxb, osal, osx, oss = (
                                (xbufb, salb, sxb, ssb) if b == 0
                                else (xbufa, sala, sxa, ssa))
                            ob = 1 - b
                            pltpu.make_async_copy(
                                alpha_hbm.at[0], albuf.at[b], sal).wait()
                            pltpu.make_async_copy(
                                xsrc8_hbm.at[0].at[rowv.at[0]], xb, sx).wait()

                            @pl.when(j + 1 < W_CHUNKS)
                            def _prefetch():
                                @pl.when(j >= 1)
                                def _ws():
                                    s_wait(oxb, oss)
                                pltpu.async_copy(
                                    alpha_hbm.at[wlin * W_CHUNKS + j + 1],
                                    albuf.at[ob], osal)
                                pltpu.async_copy(
                                    xsrc8_hbm.at[head].at[rowv.at[j + 1]],
                                    oxb, osx)

                            def edge(e, carry2):
                                wv = albuf[b, e // 8, pl.ds((e % 8) * 16, 16)]
                                ws = wv[head]
                                for q in range(8):
                                    xb[e, pl.ds(q * 16, 16)] = (
                                        ws * xb[e, pl.ds(q * 16, 16)])
                                return carry2
                            lax.fori_loop(0, CHUNK, edge, 0)
                            pltpu.async_copy(
                                xb, acc.at[colv.at[j]], ss, add=True)
                        return carry
                    lax.fori_loop(0, W_CHUNKS // 2, body, 0)
                    s_wait(xbufa, ssa)
                    s_wait(xbufb, ssb)
                plsc.subcore_barrier()
                pltpu.sync_copy(
                    acc.at[pl.ds(s * ROWS_PER_TILE, ROWS_PER_TILE)],
                    out_hbm.at[head, pl.ds(s * ROWS_PER_TILE, ROWS_PER_TILE)],
                )
                plsc.subcore_barrier()

  return _sc_degree, _sc_edge_pass, _sc_gat_msg


def _aug_body(x_ref, w1_ref, b1_ref, w2_ref, b2_ref, out_ref):
    xb = x_ref[...]
    t1 = jnp.tanh(jnp.dot(xb, w1_ref[...], preferred_element_type=f32)
                  + b1_ref[...])
    t2 = jax.nn.sigmoid(jnp.dot(xb, w2_ref[...], preferred_element_type=f32)
                        + b2_ref[...])
    out_ref[...] = jnp.concatenate([xb, t1, t2], axis=1)


def _tc_aug(xp, w1, b1, w2, b2):
    return pl.pallas_call(
        _aug_body,
        grid=(GRID_M,),
        in_specs=[
            pl.BlockSpec((BM, D_IN), lambda i: (i, 0)),
            pl.BlockSpec((D_IN, D_IN), lambda i: (0, 0)),
            pl.BlockSpec((1, D_IN), lambda i: (0, 0)),
            pl.BlockSpec((D_IN, D_IN), lambda i: (0, 0)),
            pl.BlockSpec((1, D_IN), lambda i: (0, 0)),
        ],
        out_specs=pl.BlockSpec((BM, 3 * D_IN), lambda i: (i, 0)),
        out_shape=jax.ShapeDtypeStruct((N_PAD, 3 * D_IN), f32),
    )(xp, w1, b1, w2, b2)


def _bigmm_body(h_ref, w_ref, gas_ref, gad_ref, dinv_ref, scale_ref, shift_ref,
                f0_ref, f1_ref, f2_ref, hr_ref, xsrc_ref, asrc_ref, adst_ref,
                *, with_bn):
    hb = h_ref[...]
    if with_bn:
        hb = jnp.maximum(hb * scale_ref[...] + shift_ref[...], 0.0)
    p = jnp.dot(hb, w_ref[...], preferred_element_type=f32)
    dinv = dinv_ref[...]
    f0_ref[...] = p[:, :HID] * dinv
    f1_ref[...] = p[:, HID:2 * HID]
    f2_ref[...] = p[:, 2 * HID:3 * HID]
    hr_ref[...] = p[:, 3 * HID:4 * HID]
    xsrc = p[:, 4 * HID:]
    xsrc_ref[...] = xsrc
    xr = xsrc.reshape(BM, HEADS, HID)
    a_s = jnp.sum(xr * gas_ref[...][None], axis=-1)
    a_d = jnp.sum(xr * gad_ref[...][None], axis=-1)
    z = jnp.zeros((BM, 120), f32)
    asrc_ref[...] = jnp.concatenate([a_s, z], axis=1)
    adst_ref[...] = jnp.concatenate([a_d, z], axis=1)


def _tc_bigmm(h, wcat, gas, gad, dinv, scale, shift, with_bn):
    ic = h.shape[1]
    wn = wcat.shape[1]
    body = functools.partial(_bigmm_body, with_bn=with_bn)
    outsp = pl.BlockSpec((BM, HID), lambda i: (i, 0))
    outsh = jax.ShapeDtypeStruct((N_PAD, HID), f32)
    return pl.pallas_call(
        body,
        grid=(GRID_M,),
        in_specs=[
            pl.BlockSpec((BM, ic), lambda i: (i, 0)),
            pl.BlockSpec((ic, wn), lambda i: (0, 0)),
            pl.BlockSpec((HEADS, HID), lambda i: (0, 0)),
            pl.BlockSpec((HEADS, HID), lambda i: (0, 0)),
            pl.BlockSpec((BM, 1), lambda i: (i, 0)),
            pl.BlockSpec((1, ic), lambda i: (0, 0)),
            pl.BlockSpec((1, ic), lambda i: (0, 0)),
        ],
        out_specs=[outsp, outsp, outsp,
                   outsp,
                   pl.BlockSpec((BM, HEADS * HID), lambda i: (i, 0)),
                   outsp, outsp],
        out_shape=[outsh, outsh, outsh,
                   outsh,
                   jax.ShapeDtypeStruct((N_PAD, HEADS * HID), f32),
                   outsh, outsh],
    )(h, wcat, gas, gad, dinv, scale, shift)


def _assemble_body(sg_ref, ss_ref, sgin_ref, m_ref, den_ref, asrc_ref,
                   adst_ref, cmax_ref, xgp_ref, xgin_ref, hr_ref,
                   xsrc_ref, dinv_ref, dci_ref, w2_ref, bias_ref,
                   hcat_ref, stats_ref):
    i = pl.program_id(0)
    bias = bias_ref[...]
    dinv = dinv_ref[...]
    gcn = dinv * (sg_ref[...] + xgp_ref[...]) + bias[0, :HID][None]
    sage = ss_ref[...] * dci_ref[...] + hr_ref[...] + bias[0, HID:2 * HID][None]
    gpre = jnp.maximum(xgin_ref[...] + sgin_ref[...]
                       + bias[0, 3 * HID:][None], 0.0)
    gin = jnp.dot(gpre, w2_ref[...], preferred_element_type=f32) \
        + bias[1, 3 * HID:][None]
    sv = asrc_ref[...] + adst_ref[...]
    sv = jnp.where(sv > 0.0, sv, sv * 0.2)
    asl = jnp.exp(sv - cmax_ref[0, 0])[:, :HEADS]
    den8 = den_ref[...][:, :HEADS] + asl
    dinv8 = 1.0 / (den8 + 1e-16)
    mr = m_ref[...].reshape(BM, HEADS, HID)
    xr = xsrc_ref[...].reshape(BM, HEADS, HID)
    gsum = jnp.sum((mr + asl[:, :, None] * xr) * dinv8[:, :, None], axis=1)
    gat = gsum * (1.0 / HEADS) + bias[0, 2 * HID:3 * HID][None]
    hcat = jnp.concatenate([gcn, sage, gat, gin], axis=1)
    hcat_ref[...] = hcat
    rowid = i * BM + lax.broadcasted_iota(jnp.int32, (BM, 1), 0)
    hm = jnp.where(rowid < N, hcat, 0.0)
    ssum = jnp.sum(hm, axis=0, keepdims=True)
    ssq = jnp.sum(hm * hm, axis=0, keepdims=True)
    blk = jnp.concatenate([ssum, ssq, jnp.zeros((6, 4 * HID), f32)], axis=0)

    @pl.when(i == 0)
    def _init():
        stats_ref[...] = blk

    @pl.when(i != 0)
    def _acc():
        stats_ref[...] = stats_ref[...] + blk


def _tc_assemble(sg, ss, sgin, m, den, asrc, adst, cmax, xgp, xgin, hr, xsrc,
                 dinv, dci, w2, bias):
    sp = pl.BlockSpec((BM, HID), lambda i: (i, 0))
    return pl.pallas_call(
        _assemble_body,
        grid=(GRID_M,),
        in_specs=[
            sp, sp, sp,
            pl.BlockSpec((BM, HEADS * HID), lambda i: (i, 0)),
            sp, sp, sp,
            pl.BlockSpec((1, 16), lambda i: (0, 0)),
            sp, sp, sp,
            pl.BlockSpec((BM, HEADS * HID), lambda i: (i, 0)),
            pl.BlockSpec((BM, 1), lambda i: (i, 0)),
            pl.BlockSpec((BM, 1), lambda i: (i, 0)),
            pl.BlockSpec((HID, HID), lambda i: (0, 0)),
            pl.BlockSpec((2, 4 * HID), lambda i: (0, 0)),
        ],
        out_specs=[
            pl.BlockSpec((BM, 4 * HID), lambda i: (i, 0)),
            pl.BlockSpec((8, 4 * HID), lambda i: (0, 0)),
        ],
        out_shape=[
            jax.ShapeDtypeStruct((N_PAD, 4 * HID), f32),
            jax.ShapeDtypeStruct((8, 4 * HID), f32),
        ],
    )(sg, ss, sgin, m, den, asrc, adst, cmax, xgp, xgin, hr, xsrc,
      dinv, dci, w2, bias)


def _final_body(h_ref, scale_ref, shift_ref, w_ref, b_ref, out_ref):
    hb = jnp.maximum(h_ref[...] * scale_ref[...] + shift_ref[...], 0.0)
    z = jnp.dot(hb, w_ref[...], preferred_element_type=f32) + b_ref[...]
    colid = lax.broadcasted_iota(jnp.int32, (BM, 128), 1)
    zm = jnp.where(colid < N_CLS, z, -jnp.inf)
    m = jnp.max(zm, axis=1, keepdims=True)
    lse = jnp.log(jnp.sum(jnp.exp(zm - m), axis=1, keepdims=True))
    out_ref[...] = (z - m - lse)[:, :N_CLS]


def _tc_final(h, scale, shift, w, b):
    return pl.pallas_call(
        _final_body,
        grid=(GRID_M,),
        in_specs=[
            pl.BlockSpec((BM, 4 * HID), lambda i: (i, 0)),
            pl.BlockSpec((1, 4 * HID), lambda i: (0, 0)),
            pl.BlockSpec((1, 4 * HID), lambda i: (0, 0)),
            pl.BlockSpec((4 * HID, 128), lambda i: (0, 0)),
            pl.BlockSpec((1, 128), lambda i: (0, 0)),
        ],
        out_specs=pl.BlockSpec((BM, N_CLS), lambda i: (i, 0)),
        out_shape=jax.ShapeDtypeStruct((N_PAD, N_CLS), f32),
    )(h, scale, shift, w, b)


def kernel(x, edge_index, params):
    p = params
    row = edge_index[0].astype(jnp.int32)
    col = edge_index[1].astype(jnp.int32)
    pad_idx = N + (jnp.arange(E_PAD - E, dtype=jnp.int32) % (N_PAD - N))
    rowp = jnp.concatenate([row, pad_idx])
    colp = jnp.concatenate([col, pad_idx])
    rows32 = rowp.reshape(NW, W_CHUNKS, CHUNK)
    cols32 = colp.reshape(NW, W_CHUNKS, CHUNK)

    z128 = jnp.zeros((ROWS_PER_TILE, 128), f32)
    o128 = jnp.ones((CHUNK, 128), f32)
    sc_degree, sc_edge_pass, sc_gat_msg = _sc_kernels()

    degp = sc_degree(cols32, o128, z128)
    deg = degp[0, :, 0] + degp[1, :, 0]
    dinv = (deg + 1.0) ** -0.5
    dci = 1.0 / jnp.maximum(deg, 1.0)
    dinv2 = dinv[:, None]
    dci2 = dci[:, None]

    xp = jnp.pad(x, ((0, N_PAD - N), (0, 0)))
    h = _tc_aug(xp, p['aug_W1'], p['aug_b1'][None], p['aug_W2'],
                p['aug_b2'][None])

    scale = jnp.ones((1, 3 * D_IN), f32)
    shift = jnp.zeros((1, 3 * D_IN), f32)
    for i in range(2):
        wcat = jnp.concatenate(
            [p[f'gcn_W{i}'], p[f'sage_Wl{i}'], p[f'gin_W1{i}'],
             p[f'sage_Wr{i}'], p[f'gat_W{i}']], axis=1)
        f0, f1, f2, hr, xsrc, asrc, adst = _tc_bigmm(
            h, wcat, p[f'gat_as{i}'], p[f'gat_ad{i}'], dinv2, scale, shift,
            with_bn=(i > 0))
        cmax = jnp.max(asrc[:N, :8]) + jnp.max(adst[:N, :8])
        cmax = jnp.maximum(cmax, 0.0)
        cmax16 = jnp.full((1, 16), cmax, f32)

        s, den, alpha = sc_edge_pass(f0, f1, f2, asrc, adst,
                                      rows32, cols32, cmax16, z128)
        xsrc8 = xsrc.reshape(N_PAD, HEADS, HID).transpose(1, 0, 2)
        m8 = sc_gat_msg(xsrc8, alpha, rows32, cols32, z128)
        m = m8.transpose(1, 0, 2).reshape(N_PAD, HEADS * HID)

        bias = jnp.stack([
            jnp.concatenate([p[f'gcn_b{i}'], p[f'sage_b{i}'], p[f'gat_b{i}'],
                             p[f'gin_b1{i}']]),
            jnp.concatenate([jnp.zeros((3 * HID,), f32), p[f'gin_b2{i}']]),
        ])
        hcat, stats = _tc_assemble(
            s[0, 0] + s[0, 1], s[1, 0] + s[1, 1], s[2, 0] + s[2, 1],
            m, den[0] + den[1], asrc, adst, cmax16, f0, f2, hr, xsrc,
            dinv2, dci2, p[f'gin_W2{i}'], bias)
        mu = stats[0] / N
        var = stats[1] / N - mu * mu
        scale = (p[f'bn_g{i}'] / jnp.sqrt(var + 1e-5))[None]
        shift = (p[f'bn_b{i}'] - mu * scale[0])[None]
        h = hcat

    wout = jnp.pad(p['out_W'], ((0, 0), (0, 128 - N_CLS)))
    bout = jnp.pad(p['out_b'], ((0, 128 - N_CLS)))[None]
    out = _tc_final(h, scale, shift, wout, bout)
    return out[:N]


# speedup vs baseline: 17.5700x; 1.0209x over previous
import functools

import jax
import jax.numpy as jnp
from jax import lax
from jax.experimental import pallas as pl
from jax.experimental.pallas import tpu as pltpu
from jax.experimental.pallas import tpu_sc as plsc

N = 10000
E = 160000
D_IN = 256
HID = 128
HEADS = 8
N_CLS = 40

N_PAD = 10240
E_PAD = 163840
DUMMY = N_PAD - 1
NW = 32
CHUNK = 128
W_CHUNKS = E_PAD // (NW * CHUNK)
ROWS_PER_TILE = N_PAD // 16

BM = 512
GRID_M = N_PAD // BM

f32 = jnp.float32


def _zero_acc(zrow_hbm, acc, s):
    pltpu.sync_copy(zrow_hbm, acc.at[pl.ds(s * ROWS_PER_TILE, ROWS_PER_TILE)])


@functools.cache
def _sc_kernels():
  mesh = plsc.VectorSubcoreMesh(core_axis_name="c", subcore_axis_name="s")

  @functools.partial(
      pl.kernel, mesh=mesh,
      out_type=jax.ShapeDtypeStruct((2, N_PAD, 128), f32),
      scratch_types=[
          pltpu.VMEM((W_CHUNKS, CHUNK), jnp.int32),
          pltpu.VMEM((CHUNK, 128), f32),
          pltpu.VMEM_SHARED((N_PAD, 128), f32),
      ],
  )
  def _sc_degree(cols_hbm, ones_hbm, zrow_hbm, out_hbm, colv, obuf, acc):
    c = lax.axis_index("c")
    s = lax.axis_index("s")
    w = s * 2 + c
    pltpu.sync_copy(cols_hbm.at[w], colv)
    pltpu.sync_copy(ones_hbm, obuf)
    _zero_acc(zrow_hbm, acc, s)
    plsc.subcore_barrier()

    def body(j, carry):
        pltpu.sync_copy(obuf, acc.at[colv.at[j]], add=True)
        return carry
    lax.fori_loop(0, W_CHUNKS, body, 0)
    plsc.subcore_barrier()
    pltpu.sync_copy(
        acc.at[pl.ds(s * ROWS_PER_TILE, ROWS_PER_TILE)],
        out_hbm.at[c, pl.ds(s * ROWS_PER_TILE, ROWS_PER_TILE)],
    )

  @functools.partial(
      pl.kernel, mesh=mesh,
      out_type=[
          jax.ShapeDtypeStruct((3, N_PAD, 128), f32),
          jax.ShapeDtypeStruct((N_PAD, 128), f32),
          jax.ShapeDtypeStruct((NW * W_CHUNKS, 16, 128), f32),
      ],
      scratch_types=[
          pltpu.VMEM((W_CHUNKS, CHUNK), jnp.int32),
          pltpu.VMEM((W_CHUNKS, CHUNK), jnp.int32),
          pltpu.VMEM((CHUNK, 128), f32),
          pltpu.VMEM((CHUNK, 128), f32),
          pltpu.VMEM((16, 128), f32),
          pltpu.VMEM((1, 16), f32),
          pltpu.VMEM_SHARED((N_PAD, 128), f32),
          pltpu.SemaphoreType.DMA,
          pltpu.SemaphoreType.DMA,
          pltpu.SemaphoreType.DMA,
          pltpu.SemaphoreType.DMA,
      ],
  )
  def _sc_edge_pass(f0_hbm, f1_hbm, f2_hbm, asrc_hbm, adst_hbm,
                    rows_hbm, cols_hbm, cmax_hbm, zrow_hbm,
                    feat_out, den_out, alpha_out,
                    rowv, colv, bufa, bufb, albuf, cbuf, acc,
                    sga, sgb, ssa, ssb):
    c = lax.axis_index("c")
    s = lax.axis_index("s")
    pltpu.sync_copy(cmax_hbm, cbuf)

    def s_start(j, buf, sem):
        pltpu.async_copy(buf, acc.at[colv.at[j]], sem, add=True)

    def s_wait(buf, sem):
        pltpu.make_async_copy(buf, acc.at[colv.at[0]], sem).wait()

    @pl.when(c == 0)
    def _feats():
        for p, f_hbm in enumerate((f0_hbm, f1_hbm, f2_hbm)):
            _zero_acc(zrow_hbm, acc, s)
            plsc.subcore_barrier()
            for hh in range(2):
                wlin = 2 * s + hh
                pltpu.sync_copy(rows_hbm.at[wlin], rowv)
                pltpu.sync_copy(cols_hbm.at[wlin], colv)
                pltpu.async_copy(f_hbm.at[rowv.at[0]], bufa, sga)

                def body(j2, carry):
                    for b in range(2):
                        j = 2 * j2 + b
                        buf, sg, ss = (bufa, sga, ssa) if b == 0 \
                            else (bufb, sgb, ssb)
                        obuf, osg, oss = (bufb, sgb, ssb) if b == 0 \
                            else (bufa, sga, ssa)
                        pltpu.make_async_copy(
                            f_hbm.at[rowv.at[0]], buf, sg).wait()

                        @pl.when(j + 1 < W_CHUNKS)
                        def _prefetch():
                            @pl.when(j >= 1)
                            def _ws():
                                s_wait(obuf, oss)
                            pltpu.async_copy(
                                f_hbm.at[rowv.at[j + 1]], obuf, osg)

                        s_start(j, buf, ss)
                    return carry
                lax.fori_loop(0, W_CHUNKS // 2, body, 0)
                s_wait(bufa, ssa)
                s_wait(bufb, ssb)
            plsc.subcore_barrier()
            pltpu.sync_copy(
                acc.at[pl.ds(s * ROWS_PER_TILE, ROWS_PER_TILE)],
                feat_out.at[p, pl.ds(s * ROWS_PER_TILE, ROWS_PER_TILE)],
            )
            plsc.subcore_barrier()

    @pl.when(c == 1)
    def _alpha():
        _zero_acc(zrow_hbm, acc, s)
        plsc.subcore_barrier()
        for hh in range(2):
            if hh > 0:
                s_wait(bufb, ssb)
            wlin = 2 * s + hh
            pltpu.sync_copy(rows_hbm.at[wlin], rowv)
            pltpu.sync_copy(cols_hbm.at[wlin], colv)

            def body_a(j, carry):
                @pl.when(j >= 1)
                def _ws():
                    s_wait(bufb, ssb)
                pltpu.async_copy(asrc_hbm.at[rowv.at[j]], bufa, sga)
                pltpu.async_copy(adst_hbm.at[colv.at[j]], bufb, sgb)
                pltpu.make_async_copy(asrc_hbm.at[rowv.at[0]], bufa, sga).wait()
                pltpu.make_async_copy(adst_hbm.at[colv.at[0]], bufb, sgb).wait()
                cv = cbuf[0]

                def edge(e, carry2):
                    sv = bufa[e, pl.ds(0, 16)] + bufb[e, pl.ds(0, 16)]
                    sv = jnp.where(sv > 0.0, sv, sv * 0.2)
                    al = jnp.exp(sv - cv)
                    bufb[e, pl.ds(0, 16)] = al
                    albuf[e // 8, pl.ds((e % 8) * 16, 16)] = al
                    return carry2
                lax.fori_loop(0, CHUNK, edge, 0)
                pltpu.sync_copy(albuf, alpha_out.at[wlin * W_CHUNKS + j])
                s_start(j, bufb, ssb)
                return carry
            lax.fori_loop(0, W_CHUNKS, body_a, 0)
        s_wait(bufb, ssb)
        plsc.subcore_barrier()
        pltpu.sync_copy(
            acc.at[pl.ds(s * ROWS_PER_TILE, ROWS_PER_TILE)],
            den_out.at[pl.ds(s * ROWS_PER_TILE, ROWS_PER_TILE)],
        )

  @functools.partial(
      pl.kernel, mesh=mesh,
      out_type=jax.ShapeDtypeStruct((HEADS, N_PAD, HID), f32),
      scratch_types=[
          pltpu.VMEM((W_CHUNKS, CHUNK), jnp.int32),
          pltpu.VMEM((W_CHUNKS, CHUNK), jnp.int32),
          pltpu.VMEM((2, 16, 128), f32),
          pltpu.VMEM((CHUNK, HID), f32),
          pltpu.VMEM((CHUNK, HID), f32),
          pltpu.VMEM_SHARED((N_PAD, HID), f32),
          pltpu.SemaphoreType.DMA,
          pltpu.SemaphoreType.DMA,
          pltpu.SemaphoreType.DMA,
          pltpu.SemaphoreType.DMA,
          pltpu.SemaphoreType.DMA,
          pltpu.SemaphoreType.DMA,
      ],
  )
  def _sc_gat_msg(xsrc8_hbm, alpha_hbm, rows_hbm, cols_hbm, zrow_hbm,
                  out_hbm, rowv, colv, albuf, xbufa, xbufb, acc,
                  sala, salb, sxa, sxb, ssa, ssb):
    c = lax.axis_index("c")
    s = lax.axis_index("s")

    def s_wait(buf, sem):
        pltpu.make_async_copy(buf, acc.at[colv.at[0]], sem).wait()

    for cval in range(2):
        @pl.when(c == cval)
        def _per_core():
            for h in range(4):
                head = cval * 4 + h
                _zero_acc(zrow_hbm, acc, s)
                plsc.subcore_barrier()
                for hh in range(2):
                    wlin = 2 * s + hh
                    pltpu.sync_copy(rows_hbm.at[wlin], rowv)
                    pltpu.sync_copy(cols_hbm.at[wlin], colv)
                    pltpu.async_copy(
                        alpha_hbm.at[wlin * W_CHUNKS], albuf.at[0], sala)
                    pltpu.async_copy(
                        xsrc8_hbm.at[head].at[rowv.at[0]], xbufa, sxa)

                    def body(j2, carry):
                        for b in range(2):
                            j = 2 * j2 + b
                            xb, sal, sx, ss = (
                                (xbufa, sala, sxa, ssa) if b == 0
                                else (xbufb, salb, sxb, ssb))
                            oxb, osal, osx, oss = (
                                (xbufb, salb, sxb, ssb) if b == 0
                                else (xbufa, sala, sxa, ssa))
                            ob = 1 - b
                            pltpu.make_async_copy(
                                alpha_hbm.at[0], albuf.at[b], sal).wait()
                            pltpu.make_async_copy(
                                xsrc8_hbm.at[0].at[rowv.at[0]], xb, sx).wait()

                            @pl.when(j + 1 < W_CHUNKS)
                            def _prefetch():
                                @pl.when(j >= 1)
                                def _ws():
                                    s_wait(oxb, oss)
                                pltpu.async_copy(
                                    alpha_hbm.at[wlin * W_CHUNKS + j + 1],
                                    albuf.at[ob], osal)
                                pltpu.async_copy(
                                    xsrc8_hbm.at[head].at[rowv.at[j + 1]],
                                    oxb, osx)

                            def edge(e, carry2):
                                wv = albuf[b, e // 8, pl.ds((e % 8) * 16, 16)]
                                ws = wv[head]
                                for q in range(8):
                                    xb[e, pl.ds(q * 16, 16)] = (
                                        ws * xb[e, pl.ds(q * 16, 16)])
                                return carry2
                            lax.fori_loop(0, CHUNK, edge, 0)
                            pltpu.async_copy(
                                xb, acc.at[colv.at[j]], ss, add=True)
                        return carry
                    lax.fori_loop(0, W_CHUNKS // 2, body, 0)
                    s_wait(xbufa, ssa)
                    s_wait(xbufb, ssb)
                plsc.subcore_barrier()
                pltpu.sync_copy(
                    acc.at[pl.ds(s * ROWS_PER_TILE, ROWS_PER_TILE)],
                    out_hbm.at[head, pl.ds(s * ROWS_PER_TILE, ROWS_PER_TILE)],
                )
                plsc.subcore_barrier()

  return _sc_degree, _sc_edge_pass, _sc_gat_msg


def _aug_body(x_ref, w1_ref, b1_ref, w2_ref, b2_ref, out_ref):
    xb = x_ref[...]
    t1 = jnp.tanh(jnp.dot(xb, w1_ref[...], preferred_element_type=f32)
                  + b1_ref[...])
    t2 = jax.nn.sigmoid(jnp.dot(xb, w2_ref[...], preferred_element_type=f32)
                        + b2_ref[...])
    out_ref[...] = jnp.concatenate([xb, t1, t2], axis=1)


def _tc_aug(xp, w1, b1, w2, b2):
    return pl.pallas_call(
        _aug_body,
        grid=(GRID_M,),
        in_specs=[
            pl.BlockSpec((BM, D_IN), lambda i: (i, 0)),
            pl.BlockSpec((D_IN, D_IN), lambda i: (0, 0)),
            pl.BlockSpec((1, D_IN), lambda i: (0, 0)),
            pl.BlockSpec((D_IN, D_IN), lambda i: (0, 0)),
            pl.BlockSpec((1, D_IN), lambda i: (0, 0)),
        ],
        out_specs=pl.BlockSpec((BM, 3 * D_IN), lambda i: (i, 0)),
        out_shape=jax.ShapeDtypeStruct((N_PAD, 3 * D_IN), f32),
    )(xp, w1, b1, w2, b2)


def _bigmm_body(h_ref, w_ref, gas_ref, gad_ref, dinv_ref, scale_ref, shift_ref,
                f0_ref, f1_ref, f2_ref, hr_ref, xsrc_ref, asrc_ref, adst_ref,
                *, with_bn):
    hb = h_ref[...]
    if with_bn:
        hb = jnp.maximum(hb * scale_ref[...] + shift_ref[...], 0.0)
    p = jnp.dot(hb, w_ref[...], preferred_element_type=f32)
    dinv = dinv_ref[...]
    f0_ref[...] = p[:, :HID] * dinv
    f1_ref[...] = p[:, HID:2 * HID]
    f2_ref[...] = p[:, 2 * HID:3 * HID]
    hr_ref[...] = p[:, 3 * HID:4 * HID]
    xsrc = p[:, 4 * HID:]
    xsrc_ref[...] = xsrc
    xr = xsrc.reshape(BM, HEADS, HID)
    a_s = jnp.sum(xr * gas_ref[...][None], axis=-1)
    a_d = jnp.sum(xr * gad_ref[...][None], axis=-1)
    z = jnp.zeros((BM, 120), f32)
    asrc_ref[...] = jnp.concatenate([a_s, z], axis=1)
    adst_ref[...] = jnp.concatenate([a_d, z], axis=1)


def _tc_bigmm(h, wcat, gas, gad, dinv, scale, shift, with_bn):
    ic = h.shape[1]
    wn = wcat.shape[1]
    body = functools.partial(_bigmm_body, with_bn=with_bn)
    outsp = pl.BlockSpec((BM, HID), lambda i: (i, 0))
    outsh = jax.ShapeDtypeStruct((N_PAD, HID), f32)
    return pl.pallas_call(
        body,
        grid=(GRID_M,),
        in_specs=[
            pl.BlockSpec((BM, ic), lambda i: (i, 0)),
            pl.BlockSpec((ic, wn), lambda i: (0, 0)),
            pl.BlockSpec((HEADS, HID), lambda i: (0, 0)),
            pl.BlockSpec((HEADS, HID), lambda i: (0, 0)),
            pl.BlockSpec((BM, 1), lambda i: (i, 0)),
            pl.BlockSpec((1, ic), lambda i: (0, 0)),
            pl.BlockSpec((1, ic), lambda i: (0, 0)),
        ],
        out_specs=[outsp, outsp, outsp,
                   outsp,
                   pl.BlockSpec((BM, HEADS * HID), lambda i: (i, 0)),
                   outsp, outsp],
        out_shape=[outsh, outsh, outsh,
                   outsh,
                   jax.ShapeDtypeStruct((N_PAD, HEADS * HID), f32),
                   outsh, outsh],
    )(h, wcat, gas, gad, dinv, scale, shift)


def _assemble_body(sg_ref, ss_ref, sgin_ref, m_ref, den_ref, asrc_ref,
                   adst_ref, cmax_ref, xgp_ref, xgin_ref, hr_ref,
                   xsrc_ref, dinv_ref, dci_ref, w2_ref, bias_ref,
                   hcat_ref, stats_ref):
    i = pl.program_id(0)
    bias = bias_ref[...]
    dinv = dinv_ref[...]
    gcn = dinv * (sg_ref[...] + xgp_ref[...]) + bias[0, :HID][None]
    sage = ss_ref[...] * dci_ref[...] + hr_ref[...] + bias[0, HID:2 * HID][None]
    gpre = jnp.maximum(xgin_ref[...] + sgin_ref[...]
                       + bias[0, 3 * HID:][None], 0.0)
    gin = jnp.dot(gpre, w2_ref[...], preferred_element_type=f32) \
        + bias[1, 3 * HID:][None]
    sv = asrc_ref[...] + adst_ref[...]
    sv = jnp.where(sv > 0.0, sv, sv * 0.2)
    asl = jnp.exp(sv - cmax_ref[0, 0])[:, :HEADS]
    den8 = den_ref[...][:, :HEADS] + asl
    dinv8 = 1.0 / (den8 + 1e-16)
    mr = m_ref[...].reshape(BM, HEADS, HID)
    xr = xsrc_ref[...].reshape(BM, HEADS, HID)
    gsum = jnp.sum((mr + asl[:, :, None] * xr) * dinv8[:, :, None], axis=1)
    gat = gsum * (1.0 / HEADS) + bias[0, 2 * HID:3 * HID][None]
    hcat = jnp.concatenate([gcn, sage, gat, gin], axis=1)
    hcat_ref[...] = hcat
    rowid = i * BM + lax.broadcasted_iota(jnp.int32, (BM, 1), 0)
    hm = jnp.where(rowid < N, hcat, 0.0)
    ssum = jnp.sum(hm, axis=0, keepdims=True)
    ssq = jnp.sum(hm * hm, axis=0, keepdims=True)
    blk = jnp.concatenate([ssum, ssq, jnp.zeros((6, 4 * HID), f32)], axis=0)

    @pl.when(i == 0)
    def _init():
        stats_ref[...] = blk

    @pl.when(i != 0)
    def _acc():
        stats_ref[...] = stats_ref[...] + blk


def _tc_assemble(sg, ss, sgin, m, den, asrc, adst, cmax, xgp, xgin, hr, xsrc,
                 dinv, dci, w2, bias):
    sp = pl.BlockSpec((BM, HID), lambda i: (i, 0))
    return pl.pallas_call(
        _assemble_body,
        grid=(GRID_M,),
        in_specs=[
            sp, sp, sp,
            pl.BlockSpec((BM, HEADS * HID), lambda i: (i, 0)),
            sp, sp, sp,
            pl.BlockSpec((1, 16), lambda i: (0, 0)),
            sp, sp, sp,
            pl.BlockSpec((BM, HEADS * HID), lambda i: (i, 0)),
            pl.BlockSpec((BM, 1), lambda i: (i, 0)),
            pl.BlockSpec((BM, 1), lambda i: (i, 0)),
            pl.BlockSpec((HID, HID), lambda i: (0, 0)),
            pl.BlockSpec((2, 4 * HID), lambda i: (0, 0)),
        ],
        out_specs=[
            pl.BlockSpec((BM, 4 * HID), lambda i: (i, 0)),
            pl.BlockSpec((8, 4 * HID), lambda i: (0, 0)),
        ],
        out_shape=[
            jax.ShapeDtypeStruct((N_PAD, 4 * HID), f32),
            jax.ShapeDtypeStruct((8, 4 * HID), f32),
        ],
    )(sg, ss, sgin, m, den, asrc, adst, cmax, xgp, xgin, hr, xsrc,
      dinv, dci, w2, bias)


def _final_body(h_ref, scale_ref, shift_ref, w_ref, b_ref, out_ref):
    hb = jnp.maximum(h_ref[...] * scale_ref[...] + shift_ref[...], 0.0)
    z = jnp.dot(hb, w_ref[...], preferred_element_type=f32) + b_ref[...]
    colid = lax.broadcasted_iota(jnp.int32, (BM, 128), 1)
    zm = jnp.where(colid < N_CLS, z, -jnp.inf)
    m = jnp.max(zm, axis=1, keepdims=True)
    lse = jnp.log(jnp.sum(jnp.exp(zm - m), axis=1, keepdims=True))
    out_ref[...] = (z - m - lse)[:, :N_CLS]


def _tc_final(h, scale, shift, w, b):
    return pl.pallas_call(
        _final_body,
        grid=(GRID_M,),
        in_specs=[
            pl.BlockSpec((BM, 4 * HID), lambda i: (i, 0)),
            pl.BlockSpec((1, 4 * HID), lambda i: (0, 0)),
            pl.BlockSpec((1, 4 * HID), lambda i: (0, 0)),
            pl.BlockSpec((4 * HID, 128), lambda i: (0, 0)),
            pl.BlockSpec((1, 128), lambda i: (0, 0)),
        ],
        out_specs=pl.BlockSpec((BM, N_CLS), lambda i: (i, 0)),
        out_shape=jax.ShapeDtypeStruct((N_PAD, N_CLS), f32),
    )(h, scale, shift, w, b)


def kernel(x, edge_index, params):
    p = params
    row = edge_index[0].astype(jnp.int32)
    col = edge_index[1].astype(jnp.int32)
    pad_idx = N + (jnp.arange(E_PAD - E, dtype=jnp.int32) % (N_PAD - N))
    rowp = jnp.concatenate([row, pad_idx])
    colp = jnp.concatenate([col, pad_idx])
    rows32 = rowp.reshape(NW, W_CHUNKS, CHUNK)
    cols32 = colp.reshape(NW, W_CHUNKS, CHUNK)

    z128 = jnp.zeros((ROWS_PER_TILE, 128), f32)
    o128 = jnp.ones((CHUNK, 128), f32)
    sc_degree, sc_edge_pass, sc_gat_msg = _sc_kernels()

    degp = sc_degree(cols32, o128, z128)
    deg = degp[0, :, 0] + degp[1, :, 0]
    dinv = (deg + 1.0) ** -0.5
    dci = 1.0 / jnp.maximum(deg, 1.0)
    dinv2 = dinv[:, None]
    dci2 = dci[:, None]

    xp = jnp.pad(x, ((0, N_PAD - N), (0, 0)))
    h = _tc_aug(xp, p['aug_W1'], p['aug_b1'][None], p['aug_W2'],
                p['aug_b2'][None])

    scale = jnp.ones((1, 3 * D_IN), f32)
    shift = jnp.zeros((1, 3 * D_IN), f32)
    for i in range(2):
        wcat = jnp.concatenate(
            [p[f'gcn_W{i}'], p[f'sage_Wl{i}'], p[f'gin_W1{i}'],
             p[f'sage_Wr{i}'], p[f'gat_W{i}']], axis=1)
        f0, f1, f2, hr, xsrc, asrc, adst = _tc_bigmm(
            h, wcat, p[f'gat_as{i}'], p[f'gat_ad{i}'], dinv2, scale, shift,
            with_bn=(i > 0))
        cmax = jnp.max(asrc[:N, :8]) + jnp.max(adst[:N, :8])
        cmax = jnp.maximum(cmax, 0.0)
        cmax16 = jnp.full((1, 16), cmax, f32)

        s, den, alpha = sc_edge_pass(f0, f1, f2, asrc, adst,
                                      rows32, cols32, cmax16, z128)
        xsrc8 = xsrc.reshape(N_PAD, HEADS, HID).transpose(1, 0, 2)
        m8 = sc_gat_msg(xsrc8, alpha, rows32, cols32, z128)
        m = m8.transpose(1, 0, 2).reshape(N_PAD, HEADS * HID)

        bias = jnp.stack([
            jnp.concatenate([p[f'gcn_b{i}'], p[f'sage_b{i}'], p[f'gat_b{i}'],
                             p[f'gin_b1{i}']]),
            jnp.concatenate([jnp.zeros((3 * HID,), f32), p[f'gin_b2{i}']]),
        ])
        hcat, stats = _tc_assemble(
            s[0], s[1], s[2],
            m, den, asrc, adst, cmax16, f0, f2, hr, xsrc,
            dinv2, dci2, p[f'gin_W2{i}'], bias)
        mu = stats[0] / N
        var = stats[1] / N - mu * mu
        scale = (p[f'bn_g{i}'] / jnp.sqrt(var + 1e-5))[None]
        shift = (p[f'bn_b{i}'] - mu * scale[0])[None]
        h = hcat

    wout = jnp.pad(p['out_W'], ((0, 0), (0, 128 - N_CLS)))
    bout = jnp.pad(p['out_b'], ((0, 128 - N_CLS)))[None]
    out = _tc_final(h, scale, shift, wout, bout)
    return out[:N]


# speedup vs baseline: 17.6018x; 1.0018x over previous
import functools

import jax
import jax.numpy as jnp
from jax import lax
from jax.experimental import pallas as pl
from jax.experimental.pallas import tpu as pltpu
from jax.experimental.pallas import tpu_sc as plsc

N = 10000
E = 160000
D_IN = 256
HID = 128
HEADS = 8
N_CLS = 40

N_PAD = 10240
E_PAD = 163840
NW = 32
CHUNK = 128
W_CHUNKS = E_PAD // (NW * CHUNK)
ROWS_PER_TILE = N_PAD // 16

BM = 512
GRID_M = N_PAD // BM

f32 = jnp.float32


def _zero_acc(zrow_hbm, acc, s):
    pltpu.sync_copy(zrow_hbm, acc.at[pl.ds(s * ROWS_PER_TILE, ROWS_PER_TILE)])


@functools.cache
def _sc_kernels():
  mesh = plsc.VectorSubcoreMesh(core_axis_name="c", subcore_axis_name="s")

  @functools.partial(
      pl.kernel, mesh=mesh,
      out_type=jax.ShapeDtypeStruct((2, N_PAD, 128), f32),
      scratch_types=[
          pltpu.VMEM((W_CHUNKS, CHUNK), jnp.int32),
          pltpu.VMEM((CHUNK, 128), f32),
          pltpu.VMEM_SHARED((N_PAD, 128), f32),
      ],
  )
  def _sc_degree(cols_hbm, ones_hbm, zrow_hbm, out_hbm, colv, obuf, acc):
    c = lax.axis_index("c")
    s = lax.axis_index("s")
    w = s * 2 + c
    pltpu.sync_copy(cols_hbm.at[w], colv)
    pltpu.sync_copy(ones_hbm, obuf)
    _zero_acc(zrow_hbm, acc, s)
    plsc.subcore_barrier()

    def body(j, carry):
        pltpu.sync_copy(obuf, acc.at[colv.at[j]], add=True)
        return carry
    lax.fori_loop(0, W_CHUNKS, body, 0)
    plsc.subcore_barrier()
    pltpu.sync_copy(
        acc.at[pl.ds(s * ROWS_PER_TILE, ROWS_PER_TILE)],
        out_hbm.at[c, pl.ds(s * ROWS_PER_TILE, ROWS_PER_TILE)],
    )

  @functools.partial(
      pl.kernel, mesh=mesh,
      out_type=[
          jax.ShapeDtypeStruct((3, N_PAD, 128), f32),
          jax.ShapeDtypeStruct((N_PAD, 128), f32),
          jax.ShapeDtypeStruct((NW * W_CHUNKS, 16, 128), f32),
      ],
      scratch_types=[
          pltpu.VMEM((W_CHUNKS, CHUNK), jnp.int32),
          pltpu.VMEM((W_CHUNKS, CHUNK), jnp.int32),
          pltpu.VMEM((CHUNK, 128), f32),
          pltpu.VMEM((CHUNK, 128), f32),
          pltpu.VMEM((16, 128), f32),
          pltpu.VMEM((1, 16), f32),
          pltpu.VMEM_SHARED((N_PAD, 128), f32),
          pltpu.SemaphoreType.DMA,
          pltpu.SemaphoreType.DMA,
          pltpu.SemaphoreType.DMA,
          pltpu.SemaphoreType.DMA,
      ],
  )
  def _sc_edge_pass(f0_hbm, f1_hbm, f2_hbm, asrc_hbm, adst_hbm,
                    rows_hbm, cols_hbm, cmax_hbm, zrow_hbm,
                    feat_out, den_out, alpha_out,
                    rowv, colv, bufa, bufb, albuf, cbuf, acc,
                    sga, sgb, ssa, ssb):
    c = lax.axis_index("c")
    s = lax.axis_index("s")
    pltpu.sync_copy(cmax_hbm, cbuf)

    def s_start(j, buf, sem):
        pltpu.async_copy(buf, acc.at[colv.at[j]], sem, add=True)

    def s_wait(buf, sem):
        pltpu.make_async_copy(buf, acc.at[colv.at[0]], sem).wait()

    @pl.when(c == 0)
    def _feats():
        for p, f_hbm in enumerate((f0_hbm, f1_hbm, f2_hbm)):
            _zero_acc(zrow_hbm, acc, s)
            plsc.subcore_barrier()
            for hh in range(2):
                wlin = 2 * s + hh
                pltpu.sync_copy(rows_hbm.at[wlin], rowv)
                pltpu.sync_copy(cols_hbm.at[wlin], colv)
                pltpu.async_copy(f_hbm.at[rowv.at[0]], bufa, sga)

                def body(j2, carry):
                    for b in range(2):
                        j = 2 * j2 + b
                        buf, sg, ss = (bufa, sga, ssa) if b == 0 \
                            else (bufb, sgb, ssb)
                        obuf, osg, oss = (bufb, sgb, ssb) if b == 0 \
                            else (bufa, sga, ssa)
                        pltpu.make_async_copy(
                            f_hbm.at[rowv.at[0]], buf, sg).wait()

                        @pl.when(j + 1 < W_CHUNKS)
                        def _prefetch():
                            @pl.when(j >= 1)
                            def _ws():
                                s_wait(obuf, oss)
                            pltpu.async_copy(
                                f_hbm.at[rowv.at[j + 1]], obuf, osg)

                        s_start(j, buf, ss)
                    return carry
                lax.fori_loop(0, W_CHUNKS // 2, body, 0)
                s_wait(bufa, ssa)
                s_wait(bufb, ssb)
            plsc.subcore_barrier()
            pltpu.sync_copy(
                acc.at[pl.ds(s * ROWS_PER_TILE, ROWS_PER_TILE)],
                feat_out.at[p, pl.ds(s * ROWS_PER_TILE, ROWS_PER_TILE)],
            )
            plsc.subcore_barrier()

    @pl.when(c == 1)
    def _alpha():
        _zero_acc(zrow_hbm, acc, s)
        plsc.subcore_barrier()
        for hh in range(2):
            if hh > 0:
                s_wait(bufb, ssb)
            wlin = 2 * s + hh
            pltpu.sync_copy(rows_hbm.at[wlin], rowv)
            pltpu.sync_copy(cols_hbm.at[wlin], colv)

            def body_a(j, carry):
                @pl.when(j >= 1)
                def _ws():
                    s_wait(bufb, ssb)
                pltpu.async_copy(asrc_hbm.at[rowv.at[j]], bufa, sga)
                pltpu.async_copy(adst_hbm.at[colv.at[j]], bufb, sgb)
                pltpu.make_async_copy(asrc_hbm.at[rowv.at[0]], bufa, sga).wait()
                pltpu.make_async_copy(adst_hbm.at[colv.at[0]], bufb, sgb).wait()
                cv = cbuf[0]

                def edge(e, carry2):
                    sv = bufa[e, pl.ds(0, 16)] + bufb[e, pl.ds(0, 16)]
                    sv = jnp.where(sv > 0.0, sv, sv * 0.2)
                    al = jnp.exp(sv - cv)
                    bufb[e, pl.ds(0, 16)] = al
                    albuf[e // 8, pl.ds((e % 8) * 16, 16)] = al
                    return carry2
                lax.fori_loop(0, CHUNK, edge, 0)
                pltpu.sync_copy(albuf, alpha_out.at[wlin * W_CHUNKS + j])
                s_start(j, bufb, ssb)
                return carry
            lax.fori_loop(0, W_CHUNKS, body_a, 0)
        s_wait(bufb, ssb)
        plsc.subcore_barrier()
        pltpu.sync_copy(
            acc.at[pl.ds(s * ROWS_PER_TILE, ROWS_PER_TILE)],
            den_out.at[pl.ds(s * ROWS_PER_TILE, ROWS_PER_TILE)],
        )

  @functools.partial(
      pl.kernel, mesh=mesh,
      out_type=jax.ShapeDtypeStruct((HEADS, N_PAD, HID), f32),
      scratch_types=[
          pltpu.VMEM((W_CHUNKS, CHUNK), jnp.int32),
          pltpu.VMEM((W_CHUNKS, CHUNK), jnp.int32),
          pltpu.VMEM((2, 16, 128), f32),
          pltpu.VMEM((CHUNK, HID), f32),
          pltpu.VMEM((CHUNK, HID), f32),
          pltpu.VMEM_SHARED((N_PAD, HID), f32),
          pltpu.SemaphoreType.DMA,
          pltpu.SemaphoreType.DMA,
          pltpu.SemaphoreType.DMA,
          pltpu.SemaphoreType.DMA,
          pltpu.SemaphoreType.DMA,
          pltpu.SemaphoreType.DMA,
      ],
  )
  def _sc_gat_msg(xsrc8_hbm, alpha_hbm, rows_hbm, cols_hbm, zrow_hbm,
                  out_hbm, rowv, colv, albuf, xbufa, xbufb, acc,
                  sala, salb, sxa, sxb, ssa, ssb):
    c = lax.axis_index("c")
    s = lax.axis_index("s")

    def s_wait(buf, sem):
        pltpu.make_async_copy(buf, acc.at[colv.at[0]], sem).wait()

    for cval in range(2):
        @pl.when(c == cval)
        def _per_core():
            for h in range(4):
                head = cval * 4 + h
                _zero_acc(zrow_hbm, acc, s)
                plsc.subcore_barrier()
                for hh in range(2):
                    wlin = 2 * s + hh
                    pltpu.sync_copy(rows_hbm.at[wlin], rowv)
                    pltpu.sync_copy(cols_hbm.at[wlin], colv)
                    pltpu.async_copy(
                        alpha_hbm.at[wlin * W_CHUNKS], albuf.at[0], sala)
                    pltpu.async_copy(
                        xsrc8_hbm.at[head].at[rowv.at[0]], xbufa, sxa)

                    def body(j2, carry):
                        for b in range(2):
                            j = 2 * j2 + b
                            xb, sal, sx, ss = (
                                (xbufa, sala, sxa, ssa) if b == 0
                                else (xbufb, salb, sxb, ssb))
                            oxb, osal, osx, oss = (
                                (xbufb, salb, sxb, ssb) if b == 0
                                else (xbufa, sala, sxa, ssa))
                            ob = 1 - b
                            pltpu.make_async_copy(
                                alpha_hbm.at[0], albuf.at[b], sal).wait()
                            pltpu.make_async_copy(
                                xsrc8_hbm.at[0].at[rowv.at[0]], xb, sx).wait()

                            @pl.when(j + 1 < W_CHUNKS)
                            def _prefetch():
                                @pl.when(j >= 1)
                                def _ws():
                                    s_wait(oxb, oss)
                                pltpu.async_copy(
                                    alpha_hbm.at[wlin * W_CHUNKS + j + 1],
                                    albuf.at[ob], osal)
                                pltpu.async_copy(
                                    xsrc8_hbm.at[head].at[rowv.at[j + 1]],
                                    oxb, osx)

                            def edge(e, carry2):
                                wv = albuf[b, e // 8, pl.ds((e % 8) * 16, 16)]
                                ws = wv[head]
                                for q in range(8):
                                    xb[e, pl.ds(q * 16, 16)] = (
                                        ws * xb[e, pl.ds(q * 16, 16)])
                                return carry2
                            lax.fori_loop(0, CHUNK, edge, 0)
                            pltpu.async_copy(
                                xb, acc.at[colv.at[j]], ss, add=True)
                        return carry
                    lax.fori_loop(0, W_CHUNKS // 2, body, 0)
                    s_wait(xbufa, ssa)
                    s_wait(xbufb, ssb)
                plsc.subcore_barrier()
                pltpu.sync_copy(
                    acc.at[pl.ds(s * ROWS_PER_TILE, ROWS_PER_TILE)],
                    out_hbm.at[head, pl.ds(s * ROWS_PER_TILE, ROWS_PER_TILE)],
                )
                plsc.subcore_barrier()

  return _sc_degree, _sc_edge_pass, _sc_gat_msg


def _aug_body(x_ref, w1_ref, b1_ref, w2_ref, b2_ref, out_ref):
    xb = x_ref[...]
    t1 = jnp.tanh(jnp.dot(xb, w1_ref[...], preferred_element_type=f32)
                  + b1_ref[...])
    t2 = jax.nn.sigmoid(jnp.dot(xb, w2_ref[...], preferred_element_type=f32)
                        + b2_ref[...])
    out_ref[...] = jnp.concatenate([xb, t1, t2], axis=1)


def _tc_aug(xp, w1, b1, w2, b2):
    return pl.pallas_call(
        _aug_body,
        grid=(GRID_M,),
        in_specs=[
            pl.BlockSpec((BM, D_IN), lambda i: (i, 0)),
            pl.BlockSpec((D_IN, D_IN), lambda i: (0, 0)),
            pl.BlockSpec((1, D_IN), lambda i: (0, 0)),
            pl.BlockSpec((D_IN, D_IN), lambda i: (0, 0)),
            pl.BlockSpec((1, D_IN), lambda i: (0, 0)),
        ],
        out_specs=pl.BlockSpec((BM, 3 * D_IN), lambda i: (i, 0)),
        out_shape=jax.ShapeDtypeStruct((N_PAD, 3 * D_IN), f32),
    )(xp, w1, b1, w2, b2)


def _bigmm_body(h_ref, w_ref, gas_ref, gad_ref, dinv_ref, scale_ref, shift_ref,
                f0_ref, f1_ref, f2_ref, hr_ref, xsrc_ref, asrc_ref, adst_ref,
                *, with_bn):
    hb = h_ref[...]
    if with_bn:
        hb = jnp.maximum(hb * scale_ref[...] + shift_ref[...], 0.0)
    p = jnp.dot(hb, w_ref[...], preferred_element_type=f32)
    dinv = dinv_ref[...]
    f0_ref[...] = p[:, :HID] * dinv
    f1_ref[...] = p[:, HID:2 * HID]
    f2_ref[...] = p[:, 2 * HID:3 * HID]
    hr_ref[...] = p[:, 3 * HID:4 * HID]
    xsrc = p[:, 4 * HID:]
    xsrc_ref[...] = xsrc
    xr = xsrc.reshape(BM, HEADS, HID)
    a_s = jnp.sum(xr * gas_ref[...][None], axis=-1)
    a_d = jnp.sum(xr * gad_ref[...][None], axis=-1)
    z = jnp.zeros((BM, 120), f32)
    asrc_ref[...] = jnp.concatenate([a_s, z], axis=1)
    adst_ref[...] = jnp.concatenate([a_d, z], axis=1)


def _tc_bigmm(h, wcat, gas, gad, dinv, scale, shift, with_bn):
    ic = h.shape[1]
    wn = wcat.shape[1]
    body = functools.partial(_bigmm_body, with_bn=with_bn)
    outsp = pl.BlockSpec((BM, HID), lambda i: (i, 0))
    outsh = jax.ShapeDtypeStruct((N_PAD, HID), f32)
    return pl.pallas_call(
        body,
        grid=(GRID_M,),
        in_specs=[
            pl.BlockSpec((BM, ic), lambda i: (i, 0)),
            pl.BlockSpec((ic, wn), lambda i: (0, 0)),
            pl.BlockSpec((HEADS, HID), lambda i: (0, 0)),
            pl.BlockSpec((HEADS, HID), lambda i: (0, 0)),
            pl.BlockSpec((BM, 1), lambda i: (i, 0)),
            pl.BlockSpec((1, ic), lambda i: (0, 0)),
            pl.BlockSpec((1, ic), lambda i: (0, 0)),
        ],
        out_specs=[outsp, outsp, outsp,
                   outsp,
                   pl.BlockSpec((BM, HEADS * HID), lambda i: (i, 0)),
                   outsp, outsp],
        out_shape=[outsh, outsh, outsh,
                   outsh,
                   jax.ShapeDtypeStruct((N_PAD, HEADS * HID), f32),
                   outsh, outsh],
    )(h, wcat, gas, gad, dinv, scale, shift)


def _assemble_body(sg_ref, ss_ref, sgin_ref, m_ref, den_ref, asrc_ref,
                   adst_ref, cmax_ref, xgp_ref, xgin_ref, hr_ref,
                   xsrc_ref, dinv_ref, dci_ref, w2_ref, bias_ref,
                   hcat_ref, stats_ref):
    i = pl.program_id(0)
    bias = bias_ref[...]
    dinv = dinv_ref[...]
    gcn = dinv * (sg_ref[...] + xgp_ref[...]) + bias[0, :HID][None]
    sage = ss_ref[...] * dci_ref[...] + hr_ref[...] + bias[0, HID:2 * HID][None]
    gpre = jnp.maximum(xgin_ref[...] + sgin_ref[...]
                       + bias[0, 3 * HID:][None], 0.0)
    gin = jnp.dot(gpre, w2_ref[...], preferred_element_type=f32) \
        + bias[1, 3 * HID:][None]
    sv = asrc_ref[...] + adst_ref[...]
    sv = jnp.where(sv > 0.0, sv, sv * 0.2)
    asl = jnp.exp(sv - cmax_ref[0, 0])[:, :HEADS]
    den8 = den_ref[...][:, :HEADS] + asl
    dinv8 = 1.0 / (den8 + 1e-16)
    mr = m_ref[...].reshape(BM, HEADS, HID)
    xr = xsrc_ref[...].reshape(BM, HEADS, HID)
    gsum = jnp.sum((mr + asl[:, :, None] * xr) * dinv8[:, :, None], axis=1)
    gat = gsum * (1.0 / HEADS) + bias[0, 2 * HID:3 * HID][None]
    hcat = jnp.concatenate([gcn, sage, gat, gin], axis=1)
    hcat_ref[...] = hcat
    rowid = i * BM + lax.broadcasted_iota(jnp.int32, (BM, 1), 0)
    hm = jnp.where(rowid < N, hcat, 0.0)
    ssum = jnp.sum(hm, axis=0, keepdims=True)
    ssq = jnp.sum(hm * hm, axis=0, keepdims=True)
    blk = jnp.concatenate([ssum, ssq, jnp.zeros((6, 4 * HID), f32)], axis=0)

    @pl.when(i == 0)
    def _init():
        stats_ref[...] = blk

    @pl.when(i != 0)
    def _acc():
        stats_ref[...] = stats_ref[...] + blk


def _tc_assemble(sg, ss, sgin, m, den, asrc, adst, cmax, xgp, xgin, hr, xsrc,
                 dinv, dci, w2, bias):
    sp = pl.BlockSpec((BM, HID), lambda i: (i, 0))
    return pl.pallas_call(
        _assemble_body,
        grid=(GRID_M,),
        in_specs=[
            sp, sp, sp,
            pl.BlockSpec((BM, HEADS * HID), lambda i: (i, 0)),
            sp, sp, sp,
            pl.BlockSpec((1, 16), lambda i: (0, 0)),
            sp, sp, sp,
            pl.BlockSpec((BM, HEADS * HID), lambda i: (i, 0)),
            pl.BlockSpec((BM, 1), lambda i: (i, 0)),
            pl.BlockSpec((BM, 1), lambda i: (i, 0)),
            pl.BlockSpec((HID, HID), lambda i: (0, 0)),
            pl.BlockSpec((2, 4 * HID), lambda i: (0, 0)),
        ],
        out_specs=[
            pl.BlockSpec((BM, 4 * HID), lambda i: (i, 0)),
            pl.BlockSpec((8, 4 * HID), lambda i: (0, 0)),
        ],
        out_shape=[
            jax.ShapeDtypeStruct((N_PAD, 4 * HID), f32),
            jax.ShapeDtypeStruct((8, 4 * HID), f32),
        ],
    )(sg, ss, sgin, m, den, asrc, adst, cmax, xgp, xgin, hr, xsrc,
      dinv, dci, w2, bias)


def _final_body(h_ref, scale_ref, shift_ref, w_ref, b_ref, out_ref):
    hb = jnp.maximum(h_ref[...] * scale_ref[...] + shift_ref[...], 0.0)
    z = jnp.dot(hb, w_ref[...], preferred_element_type=f32) + b_ref[...]
    colid = lax.broadcasted_iota(jnp.int32, (BM, 128), 1)
    zm = jnp.where(colid < N_CLS, z, -jnp.inf)
    m = jnp.max(zm, axis=1, keepdims=True)
    lse = jnp.log(jnp.sum(jnp.exp(zm - m), axis=1, keepdims=True))
    out_ref[...] = (z - m - lse)[:, :N_CLS]


def _tc_final(h, scale, shift, w, b):
    return pl.pallas_call(
        _final_body,
        grid=(GRID_M,),
        in_specs=[
            pl.BlockSpec((BM, 4 * HID), lambda i: (i, 0)),
            pl.BlockSpec((1, 4 * HID), lambda i: (0, 0)),
            pl.BlockSpec((1, 4 * HID), lambda i: (0, 0)),
            pl.BlockSpec((4 * HID, 128), lambda i: (0, 0)),
            pl.BlockSpec((1, 128), lambda i: (0, 0)),
        ],
        out_specs=pl.BlockSpec((BM, N_CLS), lambda i: (i, 0)),
        out_shape=jax.ShapeDtypeStruct((N_PAD, N_CLS), f32),
    )(h, scale, shift, w, b)


def kernel(x, edge_index, params):
    p = params
    row = edge_index[0].astype(jnp.int32)
    col = edge_index[1].astype(jnp.int32)
    pad_idx = N + (jnp.arange(E_PAD - E, dtype=jnp.int32) % (N_PAD - N))
    rowp = jnp.concatenate([row, pad_idx])
    colp = jnp.concatenate([col, pad_idx])
    rows32 = rowp.reshape(NW, W_CHUNKS, CHUNK)
    cols32 = colp.reshape(NW, W_CHUNKS, CHUNK)

    z128 = jnp.zeros((ROWS_PER_TILE, 128), f32)
    o128 = jnp.ones((CHUNK, 128), f32)
    sc_degree, sc_edge_pass, sc_gat_msg = _sc_kernels()

    degp = sc_degree(cols32, o128, z128)
    deg = degp[0, :, 0] + degp[1, :, 0]
    dinv = (deg + 1.0) ** -0.5
    dci = 1.0 / jnp.maximum(deg, 1.0)
    dinv2 = dinv[:, None]
    dci2 = dci[:, None]

    xp = jnp.pad(x, ((0, N_PAD - N), (0, 0)))
    h = _tc_aug(xp, p['aug_W1'], p['aug_b1'][None], p['aug_W2'],
                p['aug_b2'][None])

    scale = jnp.ones((1, 3 * D_IN), f32)
    shift = jnp.zeros((1, 3 * D_IN), f32)
    for i in range(2):
        wcat = jnp.concatenate(
            [p[f'gcn_W{i}'], p[f'sage_Wl{i}'], p[f'gin_W1{i}'],
             p[f'sage_Wr{i}'], p[f'gat_W{i}']], axis=1)
        f0, f1, f2, hr, xsrc, asrc, adst = _tc_bigmm(
            h, wcat, p[f'gat_as{i}'], p[f'gat_ad{i}'], dinv2, scale, shift,
            with_bn=(i > 0))
        cmax = jnp.max(asrc[:N, :8]) + jnp.max(adst[:N, :8])
        cmax = jnp.maximum(cmax, 0.0)
        cmax16 = jnp.full((1, 16), cmax, f32)

        s, den, alpha = sc_edge_pass(f0, f1, f2, asrc, adst,
                                      rows32, cols32, cmax16, z128)
        xsrc8 = xsrc.reshape(N_PAD, HEADS, HID).transpose(1, 0, 2)
        m8 = sc_gat_msg(xsrc8, alpha, rows32, cols32, z128)
        m = m8.transpose(1, 0, 2).reshape(N_PAD, HEADS * HID)

        bias = jnp.stack([
            jnp.concatenate([p[f'gcn_b{i}'], p[f'sage_b{i}'], p[f'gat_b{i}'],
                             p[f'gin_b1{i}']]),
            jnp.concatenate([jnp.zeros((3 * HID,), f32), p[f'gin_b2{i}']]),
        ])
        hcat, stats = _tc_assemble(
            s[0], s[1], s[2],
            m, den, asrc, adst, cmax16, f0, f2, hr, xsrc,
            dinv2, dci2, p[f'gin_W2{i}'], bias)
        mu = stats[0] / N
        var = stats[1] / N - mu * mu
        scale = (p[f'bn_g{i}'] / jnp.sqrt(var + 1e-5))[None]
        shift = (p[f'bn_b{i}'] - mu * scale[0])[None]
        h = hcat

    wout = jnp.pad(p['out_W'], ((0, 0), (0, 128 - N_CLS)))
    bout = jnp.pad(p['out_b'], ((0, 128 - N_CLS)))[None]
    out = _tc_final(h, scale, shift, wout, bout)
    return out[:N]
